# Initial kernel scaffold; baseline (speedup 1.0000x reference)
import jax
import jax.numpy as jnp
from jax.experimental import pallas as pl


def kernel(x, edge_index, edge_attr, W_l, b_l, W_r, W_emb, b_emb, W_te, b_te, W_ta, b_ta, W_gate, b_gate, gamma, beta):
    raise NotImplementedError("write your pallas kernel here")


# speedup vs baseline: 7.7951x; 7.7951x over previous
import functools

import jax
import jax.numpy as jnp
from jax import lax
from jax.experimental import pallas as pl
from jax.experimental.pallas import tpu as pltpu
from jax.experimental.pallas import tpu_sc as plsc

N = 10000
E = 320000
D_IN = 128
D_OUT = 128
D_EDGE = 16
D_HALF = D_IN // 2

NC = 2
NS = 16
E_PER_TEC = E // NS
CHUNK = 80
NCHUNK = E_PER_TEC // CHUNK
N_ACC = 10240
ROWS_PER_SUB = N_ACC // NS
N_PAD = 10240
NW = NC * NS
SEL_PER_TEC = N_PAD // NW
SEL_VREGS = SEL_PER_TEC // 16
AROWS_PER_TEC = SEL_PER_TEC * D_EDGE // 128


def _edge_pass_body(x2_hbm, row_hbm, col_hbm, zrow_hbm, zdeg_hbm, neg1_hbm,
                    agg_out, deg_out, elast_out,
                    row_v, col_v, rows_v, ones_v, elast_v, sem,
                    agg_sh, deg_sh):
    c = lax.axis_index("c")
    s = lax.axis_index("s")

    pltpu.sync_copy(zrow_hbm.at[pl.ds(s * ROWS_PER_SUB, ROWS_PER_SUB)],
                    agg_sh.at[pl.ds(s * ROWS_PER_SUB, ROWS_PER_SUB)])

    @pl.when(s == 0)
    def _():
        pltpu.sync_copy(zdeg_hbm, deg_sh)

    pltpu.sync_copy(row_hbm.at[s], row_v)
    pltpu.sync_copy(col_hbm.at[s], col_v)
    pltpu.sync_copy(neg1_hbm, elast_v)
    for j in range(CHUNK // 16):
        ones_v[pl.ds(j * 16, 16)] = jnp.full((16,), 1.0, jnp.float32)

    plsc.subcore_barrier()

    lane = lax.iota(jnp.int32, 16)
    e_base = s * E_PER_TEC
    xh = x2_hbm.at[c]

    def body(i, carry):
        pltpu.async_copy(xh.at[row_v.at[i]], rows_v, sem).wait()
        pltpu.sync_copy(rows_v, agg_sh.at[col_v.at[i]], add=True)

        @pl.when(c == 0)
        def _():
            pltpu.sync_copy(ones_v, deg_sh.at[col_v.at[i]], add=True)
            base_i = e_base + i * CHUNK
            for j in range(CHUNK // 16):
                col16 = col_v[i, pl.ds(j * 16, 16)]
                e16 = (base_i + j * 16) + lane
                plsc.store_scatter(elast_v, [col16], e16)

        return carry

    lax.fori_loop(0, NCHUNK, body, 0)

    @pl.when(c == 0)
    def _():
        pltpu.sync_copy(elast_v, elast_out.at[pl.ds(s * N_PAD, N_PAD)])

    plsc.subcore_barrier()
    pltpu.sync_copy(agg_sh.at[pl.ds(s * ROWS_PER_SUB, ROWS_PER_SUB)],
                    agg_out.at[c, pl.ds(s * ROWS_PER_SUB, ROWS_PER_SUB)])

    @pl.when((s == 0) & (c == 0))
    def _():
        pltpu.sync_copy(deg_sh, deg_out)


_edge_pass = functools.partial(
    pl.kernel,
    mesh=plsc.VectorSubcoreMesh(core_axis_name="c", subcore_axis_name="s"),
    compiler_params=pltpu.CompilerParams(needs_layout_passes=False, use_tc_tiling_on_sc=False),
    out_type=[
        jax.ShapeDtypeStruct((NC, N_ACC, D_HALF), jnp.float32),
        jax.ShapeDtypeStruct((N_ACC,), jnp.float32),
        jax.ShapeDtypeStruct((NS * N_PAD,), jnp.int32),
    ],
    scratch_types=[
        pltpu.VMEM((NCHUNK, CHUNK), jnp.int32),
        pltpu.VMEM((NCHUNK, CHUNK), jnp.int32),
        pltpu.VMEM((CHUNK, D_HALF), jnp.float32),
        pltpu.VMEM((CHUNK,), jnp.float32),
        pltpu.VMEM((N_PAD,), jnp.int32),
        pltpu.SemaphoreType.DMA,
        pltpu.VMEM_SHARED((N_ACC, D_HALF), jnp.float32),
        pltpu.VMEM_SHARED((N_ACC,), jnp.float32),
    ],
)(_edge_pass_body)


def _select_pass_body(elast_hbm, attr16_hbm,
                      mask_out, attr_out,
                      tbl_v, idx_v, mask_v, eidx_v, arows_v, sem):
    c = lax.axis_index("c")
    s = lax.axis_index("s")
    wid = c * NS + s
    base = wid * SEL_PER_TEC

    for j in range(NS):
        pltpu.sync_copy(elast_hbm.at[pl.ds(j * N_PAD + base, SEL_PER_TEC)],
                        tbl_v.at[pl.ds(j * SEL_PER_TEC, SEL_PER_TEC)])

    lane = lax.iota(jnp.int32, 16)

    def combine(k, carry):
        m = tbl_v[pl.ds(k * 16, 16)]
        for j in range(1, NS):
            m = jnp.maximum(m, tbl_v[pl.ds(j * SEL_PER_TEC + k * 16, 16)])
        idx_v[pl.ds(k * 16, 16)] = jnp.maximum(m, 0)
        mask_v[pl.ds(k * 16, 16)] = jnp.where(m >= 0, 1.0, 0.0)
        return carry

    lax.fori_loop(0, SEL_VREGS, combine, 0)

    pltpu.sync_copy(mask_v, mask_out.at[pl.ds(base, SEL_PER_TEC)])

    def expand(k, carry):
        for l in range(16):
            w = k * 16 + l
            bcast = plsc.load_gather(idx_v, [jnp.zeros((16,), jnp.int32) + w])
            r = k * 2 + l // 8
            eidx_v[r, pl.ds((l % 8) * 16, 16)] = bcast * D_EDGE + lane
        return carry

    lax.fori_loop(0, SEL_VREGS, expand, 0)

    def gather(t, carry):
        pltpu.async_copy(attr16_hbm.at[eidx_v.at[t]], arows_v.at[t], sem).wait()
        return carry

    lax.fori_loop(0, AROWS_PER_TEC, gather, 0)
    pltpu.sync_copy(arows_v, attr_out.at[pl.ds(wid * AROWS_PER_TEC,
                                               AROWS_PER_TEC)])


_select_pass = functools.partial(
    pl.kernel,
    mesh=plsc.VectorSubcoreMesh(core_axis_name="c", subcore_axis_name="s"),
    compiler_params=pltpu.CompilerParams(needs_layout_passes=False, use_tc_tiling_on_sc=False),
    out_type=[
        jax.ShapeDtypeStruct((N_PAD,), jnp.float32),
        jax.ShapeDtypeStruct((N_PAD * D_EDGE // 128, 128), jnp.float32),
    ],
    scratch_types=[
        pltpu.VMEM((NS * SEL_PER_TEC,), jnp.int32),
        pltpu.VMEM((SEL_PER_TEC,), jnp.int32),
        pltpu.VMEM((SEL_PER_TEC,), jnp.float32),
        pltpu.VMEM((AROWS_PER_TEC, 128), jnp.int32),
        pltpu.VMEM((AROWS_PER_TEC, 128), jnp.float32),
        pltpu.SemaphoreType.DMA,
    ],
)(_select_pass_body)


def _mm(a, b):
    return jax.lax.dot_general(
        a, b, (((1,), (0,)), ((), ())),
        precision=jax.lax.Precision.HIGHEST,
        preferred_element_type=jnp.float32)


BM = 2000
GRID = N // BM


def _dense_a_body(x_ref, agg0_ref, agg1_ref, deg_ref,
                  attr_ref, mask_ref, WlT_ref, bl_ref, WrT_ref,
                  WembT_ref, bemb_ref, WteT_ref, bte_ref, WtaT_ref, bta_ref,
                  WgT_ref, bg_ref, o_ref, sum_ref, sq_ref):
    x = x_ref[...]
    deg = jnp.maximum(deg_ref[...], 1.0)
    h0 = agg0_ref[...] / deg
    h1 = agg1_ref[...] / deg
    WlT = WlT_ref[...]
    out = (_mm(h0, WlT[:D_HALF]) + _mm(h1, WlT[D_HALF:]) + bl_ref[...]
           + _mm(x, WrT_ref[...]))

    attr = attr_ref[...]
    emb = jnp.maximum(_mm(attr, WembT_ref[...]) + bemb_ref[...], 0.0)
    t_emb = _mm(emb, WteT_ref[...]) + bte_ref[...]
    t_attr = _mm(attr, WtaT_ref[...]) + bta_ref[...]
    t = t_emb + t_attr

    WgT = WgT_ref[...]
    logits = (_mm(out, WgT[:D_OUT]) + _mm(t, WgT[D_OUT:2 * D_OUT])
              + _mm(t_attr, WgT[2 * D_OUT:]) + bg_ref[...])
    gate = 1.0 / (1.0 + jnp.exp(-logits))
    contrib = gate * t * mask_ref[...]

    o = out + contrib
    o_ref[...] = o

    @pl.when(pl.program_id(0) == 0)
    def _():
        sum_ref[...] = jnp.zeros_like(sum_ref)
        sq_ref[...] = jnp.zeros_like(sq_ref)

    sum_ref[...] += jnp.sum(o, axis=0, keepdims=True)
    sq_ref[...] += jnp.sum(o * o, axis=0, keepdims=True)


_row_spec = pl.BlockSpec((BM, D_IN), lambda i: (i, 0))
_half_spec = pl.BlockSpec((BM, D_HALF), lambda i: (i, 0))
_col_spec = pl.BlockSpec((BM, 1), lambda i: (i, 0))


def _w_spec(r, c):
    return pl.BlockSpec((r, c), lambda i: (0, 0))


_dense_a = pl.pallas_call(
    _dense_a_body,
    grid=(GRID,),
    in_specs=[
        _row_spec, _half_spec, _half_spec, _col_spec,
        pl.BlockSpec((BM, D_EDGE), lambda i: (i, 0)), _col_spec,
        _w_spec(D_IN, D_OUT), _w_spec(1, D_OUT), _w_spec(D_IN, D_OUT),
        _w_spec(D_EDGE, D_EDGE), _w_spec(1, D_EDGE),
        _w_spec(D_EDGE, D_OUT), _w_spec(1, D_OUT),
        _w_spec(D_EDGE, D_OUT), _w_spec(1, D_OUT),
        _w_spec(3 * D_OUT, D_OUT), _w_spec(1, D_OUT),
    ],
    out_specs=[
        pl.BlockSpec((BM, D_OUT), lambda i: (i, 0)),
        _w_spec(1, D_OUT), _w_spec(1, D_OUT),
    ],
    out_shape=[
        jax.ShapeDtypeStruct((N, D_OUT), jnp.float32),
        jax.ShapeDtypeStruct((1, D_OUT), jnp.float32),
        jax.ShapeDtypeStruct((1, D_OUT), jnp.float32),
    ],
)


def _dense_b_body(o_ref, sum_ref, sq_ref, gamma_ref, beta_ref, out_ref):
    mu = sum_ref[...] * (1.0 / N)
    var = sq_ref[...] * (1.0 / N) - mu * mu
    o = o_ref[...]
    o = (o - mu) / jnp.sqrt(var + 1e-5) * gamma_ref[...] + beta_ref[...]
    out_ref[...] = jnp.maximum(o + o, 0.0)


_dense_b = pl.pallas_call(
    _dense_b_body,
    grid=(GRID,),
    in_specs=[
        pl.BlockSpec((BM, D_OUT), lambda i: (i, 0)),
        _w_spec(1, D_OUT), _w_spec(1, D_OUT),
        _w_spec(1, D_OUT), _w_spec(1, D_OUT),
    ],
    out_specs=pl.BlockSpec((BM, D_OUT), lambda i: (i, 0)),
    out_shape=jax.ShapeDtypeStruct((N, D_OUT), jnp.float32),
)


def kernel(x, edge_index, edge_attr, W_l, b_l, W_r, W_emb, b_emb,
           W_te, b_te, W_ta, b_ta, W_gate, b_gate, gamma, beta):
    row = edge_index[0].astype(jnp.int32).reshape(NS, NCHUNK, CHUNK)
    col = edge_index[1].astype(jnp.int32).reshape(NS, NCHUNK, CHUNK)
    x2 = jnp.stack([x[:, :D_HALF], x[:, D_HALF:]])
    zrow = jnp.zeros((N_ACC, D_HALF), jnp.float32)
    zdeg = jnp.zeros((N_ACC,), jnp.float32)
    neg1 = jnp.full((N_PAD,), -1, jnp.int32)
    attr16 = edge_attr.reshape(E * D_EDGE)

    agg_p, deg_p, elast = _edge_pass(x2, row, col, zrow, zdeg, neg1)
    mask, attr_rows = _select_pass(elast, attr16)

    attr_sel = attr_rows.reshape(N_PAD, D_EDGE)[:N]

    o, osum, osq = _dense_a(
        x, agg_p[0, :N], agg_p[1, :N],
        deg_p[:N].reshape(N, 1),
        attr_sel, mask[:N].reshape(N, 1),
        W_l.T, b_l.reshape(1, -1), W_r.T,
        W_emb.T, b_emb.reshape(1, -1), W_te.T, b_te.reshape(1, -1),
        W_ta.T, b_ta.reshape(1, -1),
        W_gate.T, b_gate.reshape(1, -1))
    return _dense_b(o, osum, osq, gamma.reshape(1, -1), beta.reshape(1, -1))


# speedup vs baseline: 10.9035x; 1.3988x over previous
import functools

import jax
import jax.numpy as jnp
from jax import lax
from jax.experimental import pallas as pl
from jax.experimental.pallas import tpu as pltpu
from jax.experimental.pallas import tpu_sc as plsc

N = 10000
E = 320000
D_IN = 128
D_OUT = 128
D_EDGE = 16
D_HALF = D_IN // 2

NC = 2
NS = 16
E_PER_TEC = E // NS
CHUNK = 125
NCHUNK = E_PER_TEC // CHUNK
EVREGS = E_PER_TEC // 16
N_ACC = 10240
ROWS_PER_SUB = N_ACC // NS
N_PAD = 10240
NW = NC * NS
SEL_PER_TEC = N_PAD // NW
SEL_VREGS = SEL_PER_TEC // 16
SEL_CHUNK = 80


def _edge_pass_body(x2_hbm, row_hbm, col_hbm, col2_hbm, zrow_hbm, zdeg_hbm,
                    neg1_hbm,
                    agg_out, deg_out, elast_out,
                    row_v, col_v, col2_v, rows_v0, rows_v1, ones_v, elast_v,
                    sem0, sem1, agg_sh, deg_sh):
    c = lax.axis_index("c")
    s = lax.axis_index("s")

    pltpu.sync_copy(zrow_hbm.at[pl.ds(s * ROWS_PER_SUB, ROWS_PER_SUB)],
                    agg_sh.at[pl.ds(s * ROWS_PER_SUB, ROWS_PER_SUB)])

    @pl.when(s == 0)
    def _():
        pltpu.sync_copy(zdeg_hbm, deg_sh)

    pltpu.sync_copy(row_hbm.at[s], row_v)
    pltpu.sync_copy(col_hbm.at[s], col_v)

    @pl.when(c == 0)
    def _():
        pltpu.sync_copy(col2_hbm.at[s], col2_v)
        pltpu.sync_copy(neg1_hbm, elast_v)

    for j in range(8):
        ones_v[pl.ds(j * 16, 16)] = jnp.full((16,), 1.0, jnp.float32)

    plsc.subcore_barrier()

    lane = lax.iota(jnp.int32, 16)
    e_base = s * E_PER_TEC
    xh = x2_hbm.at[c]
    ones_c = ones_v.at[pl.ds(0, CHUNK)]

    pltpu.async_copy(xh.at[row_v.at[0]], rows_v0, sem0)

    def body(k, carry):
        i0 = 2 * k
        i1 = 2 * k + 1
        pltpu.make_async_copy(xh.at[row_v.at[i0]], rows_v0, sem0).wait()
        pltpu.async_copy(xh.at[row_v.at[i1]], rows_v1, sem1)
        pltpu.sync_copy(rows_v0, agg_sh.at[col_v.at[i0]], add=True)

        @pl.when(c == 1)
        def _():
            pltpu.sync_copy(ones_c, deg_sh.at[col_v.at[i0]], add=True)

        pltpu.make_async_copy(xh.at[row_v.at[i1]], rows_v1, sem1).wait()

        @pl.when(k < NCHUNK // 2 - 1)
        def _():
            pltpu.async_copy(xh.at[row_v.at[i0 + 2]], rows_v0, sem0)

        pltpu.sync_copy(rows_v1, agg_sh.at[col_v.at[i1]], add=True)

        @pl.when(c == 1)
        def _():
            pltpu.sync_copy(ones_c, deg_sh.at[col_v.at[i1]], add=True)

        return carry

    lax.fori_loop(0, NCHUNK // 2, body, 0)

    @pl.when(c == 0)
    def _():
        def ebody(v, carry):
            col16 = col2_v[v]
            e16 = (e_base + v * 16) + lane
            plsc.store_scatter(elast_v, [col16], e16)
            return carry

        lax.fori_loop(0, EVREGS, ebody, 0)
        pltpu.sync_copy(elast_v, elast_out.at[s])

    plsc.subcore_barrier()
    pltpu.sync_copy(agg_sh.at[pl.ds(s * ROWS_PER_SUB, ROWS_PER_SUB)],
                    agg_out.at[c, pl.ds(s * ROWS_PER_SUB, ROWS_PER_SUB)])

    @pl.when((s == 0) & (c == 1))
    def _():
        pltpu.sync_copy(deg_sh, deg_out)


_edge_pass = functools.partial(
    pl.kernel,
    mesh=plsc.VectorSubcoreMesh(core_axis_name="c", subcore_axis_name="s"),
    compiler_params=pltpu.CompilerParams(
        needs_layout_passes=False, use_tc_tiling_on_sc=False),
    out_type=[
        jax.ShapeDtypeStruct((NC, N_ACC, D_HALF), jnp.float32),
        jax.ShapeDtypeStruct((N_ACC,), jnp.float32),
        jax.ShapeDtypeStruct((NS, N_PAD), jnp.int32),
    ],
    scratch_types=[
        pltpu.VMEM((NCHUNK, CHUNK), jnp.int32),
        pltpu.VMEM((NCHUNK, CHUNK), jnp.int32),
        pltpu.VMEM((EVREGS, 16), jnp.int32),
        pltpu.VMEM((CHUNK, D_HALF), jnp.float32),
        pltpu.VMEM((CHUNK, D_HALF), jnp.float32),
        pltpu.VMEM((128,), jnp.float32),
        pltpu.VMEM((N_PAD,), jnp.int32),
        pltpu.SemaphoreType.DMA,
        pltpu.SemaphoreType.DMA,
        pltpu.VMEM_SHARED((N_ACC, D_HALF), jnp.float32),
        pltpu.VMEM_SHARED((N_ACC,), jnp.float32),
    ],
)(_edge_pass_body)


def _select_pass_body(elast_hbm, attr_hbm,
                      mask_out, attr_out,
                      tbl_v, idx_v, mask_v, arows_v, sem):
    c = lax.axis_index("c")
    s = lax.axis_index("s")
    wid = c * NS + s
    base = wid * SEL_PER_TEC

    pltpu.sync_copy(elast_hbm.at[:, pl.ds(base, SEL_PER_TEC)], tbl_v)

    def combine(k, carry):
        m = tbl_v[0, pl.ds(k * 16, 16)]
        for j in range(1, NS):
            m = jnp.maximum(m, tbl_v[j, pl.ds(k * 16, 16)])
        idx_v[pl.ds(k * 16, 16)] = jnp.maximum(m, 0)
        mask_v[pl.ds(k * 16, 16)] = jnp.where(m >= 0, 1.0, 0.0)
        return carry

    lax.fori_loop(0, SEL_VREGS, combine, 0)

    pltpu.sync_copy(mask_v, mask_out.at[pl.ds(base, SEL_PER_TEC)])

    def gather(t, carry):
        pltpu.async_copy(
            attr_hbm.at[idx_v.at[pl.ds(t * SEL_CHUNK, SEL_CHUNK)]],
            arows_v.at[pl.ds(t * SEL_CHUNK, SEL_CHUNK)], sem).wait()
        return carry

    lax.fori_loop(0, SEL_PER_TEC // SEL_CHUNK, gather, 0)
    pltpu.sync_copy(arows_v, attr_out.at[pl.ds(base, SEL_PER_TEC)])


_select_pass = functools.partial(
    pl.kernel,
    mesh=plsc.VectorSubcoreMesh(core_axis_name="c", subcore_axis_name="s"),
    compiler_params=pltpu.CompilerParams(
        needs_layout_passes=False, use_tc_tiling_on_sc=False),
    out_type=[
        jax.ShapeDtypeStruct((N_PAD,), jnp.float32),
        jax.ShapeDtypeStruct((N_PAD, D_EDGE), jnp.float32),
    ],
    scratch_types=[
        pltpu.VMEM((NS, SEL_PER_TEC), jnp.int32),
        pltpu.VMEM((SEL_PER_TEC,), jnp.int32),
        pltpu.VMEM((SEL_PER_TEC,), jnp.float32),
        pltpu.VMEM((SEL_PER_TEC, D_EDGE), jnp.float32),
        pltpu.SemaphoreType.DMA,
    ],
)(_select_pass_body)


def _mm(a, b):
    return jax.lax.dot_general(
        a, b, (((1,), (0,)), ((), ())),
        precision=jax.lax.Precision.HIGHEST,
        preferred_element_type=jnp.float32)


BM = 2000
GRID = N // BM


def _dense_a_body(x_ref, agg0_ref, agg1_ref, deg_ref,
                  attr_ref, mask_ref, WlT_ref, bl_ref, WrT_ref,
                  WembT_ref, bemb_ref, WteT_ref, bte_ref, WtaT_ref, bta_ref,
                  WgT_ref, bg_ref, o_ref, sum_ref, sq_ref):
    x = x_ref[...]
    deg = jnp.maximum(deg_ref[...], 1.0)
    h0 = agg0_ref[...] / deg
    h1 = agg1_ref[...] / deg
    WlT = WlT_ref[...]
    out = (_mm(h0, WlT[:D_HALF]) + _mm(h1, WlT[D_HALF:]) + bl_ref[...]
           + _mm(x, WrT_ref[...]))

    attr = attr_ref[...]
    emb = jnp.maximum(_mm(attr, WembT_ref[...]) + bemb_ref[...], 0.0)
    t_emb = _mm(emb, WteT_ref[...]) + bte_ref[...]
    t_attr = _mm(attr, WtaT_ref[...]) + bta_ref[...]
    t = t_emb + t_attr

    WgT = WgT_ref[...]
    logits = (_mm(out, WgT[:D_OUT]) + _mm(t, WgT[D_OUT:2 * D_OUT])
              + _mm(t_attr, WgT[2 * D_OUT:]) + bg_ref[...])
    gate = 1.0 / (1.0 + jnp.exp(-logits))
    contrib = gate * t * mask_ref[...]

    o = out + contrib
    o_ref[...] = o

    @pl.when(pl.program_id(0) == 0)
    def _():
        sum_ref[...] = jnp.zeros_like(sum_ref)
        sq_ref[...] = jnp.zeros_like(sq_ref)

    sum_ref[...] += jnp.sum(o, axis=0, keepdims=True)
    sq_ref[...] += jnp.sum(o * o, axis=0, keepdims=True)


_row_spec = pl.BlockSpec((BM, D_IN), lambda i: (i, 0))
_half_spec = pl.BlockSpec((BM, D_HALF), lambda i: (i, 0))
_col_spec = pl.BlockSpec((BM, 1), lambda i: (i, 0))


def _w_spec(r, c):
    return pl.BlockSpec((r, c), lambda i: (0, 0))


_dense_a = pl.pallas_call(
    _dense_a_body,
    grid=(GRID,),
    in_specs=[
        _row_spec, _half_spec, _half_spec, _col_spec,
        pl.BlockSpec((BM, D_EDGE), lambda i: (i, 0)), _col_spec,
        _w_spec(D_IN, D_OUT), _w_spec(1, D_OUT), _w_spec(D_IN, D_OUT),
        _w_spec(D_EDGE, D_EDGE), _w_spec(1, D_EDGE),
        _w_spec(D_EDGE, D_OUT), _w_spec(1, D_OUT),
        _w_spec(D_EDGE, D_OUT), _w_spec(1, D_OUT),
        _w_spec(3 * D_OUT, D_OUT), _w_spec(1, D_OUT),
    ],
    out_specs=[
        pl.BlockSpec((BM, D_OUT), lambda i: (i, 0)),
        _w_spec(1, D_OUT), _w_spec(1, D_OUT),
    ],
    out_shape=[
        jax.ShapeDtypeStruct((N, D_OUT), jnp.float32),
        jax.ShapeDtypeStruct((1, D_OUT), jnp.float32),
        jax.ShapeDtypeStruct((1, D_OUT), jnp.float32),
    ],
)


def _dense_b_body(o_ref, sum_ref, sq_ref, gamma_ref, beta_ref, out_ref):
    mu = sum_ref[...] * (1.0 / N)
    var = sq_ref[...] * (1.0 / N) - mu * mu
    o = o_ref[...]
    o = (o - mu) / jnp.sqrt(var + 1e-5) * gamma_ref[...] + beta_ref[...]
    out_ref[...] = jnp.maximum(o + o, 0.0)


_dense_b = pl.pallas_call(
    _dense_b_body,
    grid=(GRID,),
    in_specs=[
        pl.BlockSpec((BM, D_OUT), lambda i: (i, 0)),
        _w_spec(1, D_OUT), _w_spec(1, D_OUT),
        _w_spec(1, D_OUT), _w_spec(1, D_OUT),
    ],
    out_specs=pl.BlockSpec((BM, D_OUT), lambda i: (i, 0)),
    out_shape=jax.ShapeDtypeStruct((N, D_OUT), jnp.float32),
)


def kernel(x, edge_index, edge_attr, W_l, b_l, W_r, W_emb, b_emb,
           W_te, b_te, W_ta, b_ta, W_gate, b_gate, gamma, beta):
    row = edge_index[0].astype(jnp.int32).reshape(NS, NCHUNK, CHUNK)
    col = edge_index[1].astype(jnp.int32).reshape(NS, NCHUNK, CHUNK)
    col2 = edge_index[1].astype(jnp.int32).reshape(NS, EVREGS, 16)
    x2 = jnp.stack([x[:, :D_HALF], x[:, D_HALF:]])
    zrow = jnp.zeros((N_ACC, D_HALF), jnp.float32)
    zdeg = jnp.zeros((N_ACC,), jnp.float32)
    neg1 = jnp.full((N_PAD,), -1, jnp.int32)

    agg_p, deg_p, elast = _edge_pass(x2, row, col, col2, zrow, zdeg, neg1)
    mask, attr_sel = _select_pass(elast, edge_attr)

    o, osum, osq = _dense_a(
        x, agg_p[0, :N], agg_p[1, :N],
        deg_p[:N].reshape(N, 1),
        attr_sel[:N], mask[:N].reshape(N, 1),
        W_l.T, b_l.reshape(1, -1), W_r.T,
        W_emb.T, b_emb.reshape(1, -1), W_te.T, b_te.reshape(1, -1),
        W_ta.T, b_ta.reshape(1, -1),
        W_gate.T, b_gate.reshape(1, -1))
    return _dense_b(o, osum, osq, gamma.reshape(1, -1), beta.reshape(1, -1))


# speedup vs baseline: 11.5039x; 1.0551x over previous
import functools

import jax
import jax.numpy as jnp
from jax import lax
from jax.experimental import pallas as pl
from jax.experimental.pallas import tpu as pltpu
from jax.experimental.pallas import tpu_sc as plsc

N = 10000
E = 320000
D_IN = 128
D_OUT = 128
D_EDGE = 16
D_HALF = D_IN // 2

NC = 2
NS = 16
E_PER_TEC = E // NS
CHUNK = 125
NCHUNK = E_PER_TEC // CHUNK
EVREGS = E_PER_TEC // 16
N_ACC = 10240
ROWS_PER_SUB = N_ACC // NS
N_PAD = 10240
NW = NC * NS
SEL_PER_TEC = N_PAD // NW
SEL_VREGS = SEL_PER_TEC // 16
SEL_CHUNK = 80


def _edge_pass_body(x2_hbm, row_hbm, col_hbm, col2_hbm, zrow_hbm, zdeg_hbm,
                    neg1_hbm,
                    agg_out, deg_out, elast_out,
                    row_v, col_v, col2_v, rows_v0, rows_v1, ones_v, elast_v,
                    sem0, sem1, ssem0, ssem1, agg_sh, deg_sh):
    c = lax.axis_index("c")
    s = lax.axis_index("s")

    pltpu.sync_copy(zrow_hbm.at[pl.ds(s * ROWS_PER_SUB, ROWS_PER_SUB)],
                    agg_sh.at[pl.ds(s * ROWS_PER_SUB, ROWS_PER_SUB)])

    @pl.when(s == 0)
    def _():
        pltpu.sync_copy(zdeg_hbm, deg_sh)

    pltpu.sync_copy(row_hbm.at[s], row_v)
    pltpu.sync_copy(col_hbm.at[s], col_v)

    @pl.when(c == 0)
    def _():
        pltpu.sync_copy(col2_hbm.at[s], col2_v)
        pltpu.sync_copy(neg1_hbm, elast_v)

    for j in range(8):
        ones_v[pl.ds(j * 16, 16)] = jnp.full((16,), 1.0, jnp.float32)

    plsc.subcore_barrier()

    lane = lax.iota(jnp.int32, 16)
    e_base = s * E_PER_TEC
    xh = x2_hbm.at[c]
    ones_c = ones_v.at[pl.ds(0, CHUNK)]

    pltpu.async_copy(xh.at[row_v.at[0]], rows_v0, sem0)
    pltpu.async_copy(xh.at[row_v.at[1]], rows_v1, sem1)

    def body(k, carry):
        i0 = 2 * k
        i1 = 2 * k + 1
        pltpu.make_async_copy(xh.at[row_v.at[i0]], rows_v0, sem0).wait()
        pltpu.async_copy(rows_v0, agg_sh.at[col_v.at[i0]], ssem0, add=True)

        @pl.when(c == 1)
        def _():
            pltpu.sync_copy(ones_c, deg_sh.at[col_v.at[i0]], add=True)

        pltpu.make_async_copy(xh.at[row_v.at[i1]], rows_v1, sem1).wait()
        pltpu.async_copy(rows_v1, agg_sh.at[col_v.at[i1]], ssem1, add=True)

        @pl.when(c == 1)
        def _():
            pltpu.sync_copy(ones_c, deg_sh.at[col_v.at[i1]], add=True)

        @pl.when(k < NCHUNK // 2 - 1)
        def _():
            pltpu.make_async_copy(
                rows_v0, agg_sh.at[col_v.at[i0]], ssem0).wait()
            pltpu.async_copy(xh.at[row_v.at[i0 + 2]], rows_v0, sem0)
            pltpu.make_async_copy(
                rows_v1, agg_sh.at[col_v.at[i1]], ssem1).wait()
            pltpu.async_copy(xh.at[row_v.at[i1 + 2]], rows_v1, sem1)

        return carry

    lax.fori_loop(0, NCHUNK // 2, body, 0)

    pltpu.make_async_copy(
        rows_v0, agg_sh.at[col_v.at[NCHUNK - 2]], ssem0).wait()
    pltpu.make_async_copy(
        rows_v1, agg_sh.at[col_v.at[NCHUNK - 1]], ssem1).wait()

    @pl.when(c == 0)
    def _():
        def ebody(v, carry):
            col16 = col2_v[v]
            e16 = (e_base + v * 16) + lane
            plsc.store_scatter(elast_v, [col16], e16)
            return carry

        lax.fori_loop(0, EVREGS, ebody, 0)
        pltpu.sync_copy(elast_v, elast_out.at[s])

    plsc.subcore_barrier()
    pltpu.sync_copy(agg_sh.at[pl.ds(s * ROWS_PER_SUB, ROWS_PER_SUB)],
                    agg_out.at[c, pl.ds(s * ROWS_PER_SUB, ROWS_PER_SUB)])

    @pl.when((s == 0) & (c == 1))
    def _():
        pltpu.sync_copy(deg_sh, deg_out)


_edge_pass = functools.partial(
    pl.kernel,
    mesh=plsc.VectorSubcoreMesh(core_axis_name="c", subcore_axis_name="s"),
    compiler_params=pltpu.CompilerParams(
        needs_layout_passes=False, use_tc_tiling_on_sc=False),
    out_type=[
        jax.ShapeDtypeStruct((NC, N_ACC, D_HALF), jnp.float32),
        jax.ShapeDtypeStruct((N_ACC,), jnp.float32),
        jax.ShapeDtypeStruct((NS, N_PAD), jnp.int32),
    ],
    scratch_types=[
        pltpu.VMEM((NCHUNK, CHUNK), jnp.int32),
        pltpu.VMEM((NCHUNK, CHUNK), jnp.int32),
        pltpu.VMEM((EVREGS, 16), jnp.int32),
        pltpu.VMEM((CHUNK, D_HALF), jnp.float32),
        pltpu.VMEM((CHUNK, D_HALF), jnp.float32),
        pltpu.VMEM((128,), jnp.float32),
        pltpu.VMEM((N_PAD,), jnp.int32),
        pltpu.SemaphoreType.DMA,
        pltpu.SemaphoreType.DMA,
        pltpu.SemaphoreType.DMA,
        pltpu.SemaphoreType.DMA,
        pltpu.VMEM_SHARED((N_ACC, D_HALF), jnp.float32),
        pltpu.VMEM_SHARED((N_ACC,), jnp.float32),
    ],
)(_edge_pass_body)


def _select_pass_body(elast_hbm, attr_hbm,
                      mask_out, attr_out,
                      tbl_v, idx_v, mask_v, arows_v, sem):
    c = lax.axis_index("c")
    s = lax.axis_index("s")
    wid = c * NS + s
    base = wid * SEL_PER_TEC

    pltpu.sync_copy(elast_hbm.at[:, pl.ds(base, SEL_PER_TEC)], tbl_v)

    def combine(k, carry):
        m = tbl_v[0, pl.ds(k * 16, 16)]
        for j in range(1, NS):
            m = jnp.maximum(m, tbl_v[j, pl.ds(k * 16, 16)])
        idx_v[pl.ds(k * 16, 16)] = jnp.maximum(m, 0)
        mask_v[pl.ds(k * 16, 16)] = jnp.where(m >= 0, 1.0, 0.0)
        return carry

    lax.fori_loop(0, SEL_VREGS, combine, 0)

    pltpu.sync_copy(mask_v, mask_out.at[pl.ds(base, SEL_PER_TEC)])

    def gather(t, carry):
        pltpu.async_copy(
            attr_hbm.at[idx_v.at[pl.ds(t * SEL_CHUNK, SEL_CHUNK)]],
            arows_v.at[pl.ds(t * SEL_CHUNK, SEL_CHUNK)], sem).wait()
        return carry

    lax.fori_loop(0, SEL_PER_TEC // SEL_CHUNK, gather, 0)
    pltpu.sync_copy(arows_v, attr_out.at[pl.ds(base, SEL_PER_TEC)])


_select_pass = functools.partial(
    pl.kernel,
    mesh=plsc.VectorSubcoreMesh(core_axis_name="c", subcore_axis_name="s"),
    compiler_params=pltpu.CompilerParams(
        needs_layout_passes=False, use_tc_tiling_on_sc=False),
    out_type=[
        jax.ShapeDtypeStruct((N_PAD,), jnp.float32),
        jax.ShapeDtypeStruct((N_PAD, D_EDGE), jnp.float32),
    ],
    scratch_types=[
        pltpu.VMEM((NS, SEL_PER_TEC), jnp.int32),
        pltpu.VMEM((SEL_PER_TEC,), jnp.int32),
        pltpu.VMEM((SEL_PER_TEC,), jnp.float32),
        pltpu.VMEM((SEL_PER_TEC, D_EDGE), jnp.float32),
        pltpu.SemaphoreType.DMA,
    ],
)(_select_pass_body)


def _mm(a, b):
    return jax.lax.dot_general(
        a, b, (((1,), (0,)), ((), ())),
        precision=jax.lax.Precision.HIGHEST,
        preferred_element_type=jnp.float32)


BM = 2000
GRID = N // BM


def _dense_a_body(x_ref, agg0_ref, agg1_ref, deg_ref,
                  attr_ref, mask_ref, WlT_ref, bl_ref, WrT_ref,
                  WembT_ref, bemb_ref, WteT_ref, bte_ref, WtaT_ref, bta_ref,
                  WgT_ref, bg_ref, o_ref, sum_ref, sq_ref):
    x = x_ref[...]
    deg = jnp.maximum(deg_ref[...], 1.0)
    h0 = agg0_ref[...] / deg
    h1 = agg1_ref[...] / deg
    WlT = WlT_ref[...]
    out = (_mm(h0, WlT[:D_HALF]) + _mm(h1, WlT[D_HALF:]) + bl_ref[...]
           + _mm(x, WrT_ref[...]))

    attr = attr_ref[...]
    emb = jnp.maximum(_mm(attr, WembT_ref[...]) + bemb_ref[...], 0.0)
    t_emb = _mm(emb, WteT_ref[...]) + bte_ref[...]
    t_attr = _mm(attr, WtaT_ref[...]) + bta_ref[...]
    t = t_emb + t_attr

    WgT = WgT_ref[...]
    logits = (_mm(out, WgT[:D_OUT]) + _mm(t, WgT[D_OUT:2 * D_OUT])
              + _mm(t_attr, WgT[2 * D_OUT:]) + bg_ref[...])
    gate = 1.0 / (1.0 + jnp.exp(-logits))
    contrib = gate * t * mask_ref[...]

    o = out + contrib
    o_ref[...] = o

    @pl.when(pl.program_id(0) == 0)
    def _():
        sum_ref[...] = jnp.zeros_like(sum_ref)
        sq_ref[...] = jnp.zeros_like(sq_ref)

    sum_ref[...] += jnp.sum(o, axis=0, keepdims=True)
    sq_ref[...] += jnp.sum(o * o, axis=0, keepdims=True)


_row_spec = pl.BlockSpec((BM, D_IN), lambda i: (i, 0))
_half_spec = pl.BlockSpec((BM, D_HALF), lambda i: (i, 0))
_col_spec = pl.BlockSpec((BM, 1), lambda i: (i, 0))


def _w_spec(r, c):
    return pl.BlockSpec((r, c), lambda i: (0, 0))


_dense_a = pl.pallas_call(
    _dense_a_body,
    grid=(GRID,),
    in_specs=[
        _row_spec, _half_spec, _half_spec, _col_spec,
        pl.BlockSpec((BM, D_EDGE), lambda i: (i, 0)), _col_spec,
        _w_spec(D_IN, D_OUT), _w_spec(1, D_OUT), _w_spec(D_IN, D_OUT),
        _w_spec(D_EDGE, D_EDGE), _w_spec(1, D_EDGE),
        _w_spec(D_EDGE, D_OUT), _w_spec(1, D_OUT),
        _w_spec(D_EDGE, D_OUT), _w_spec(1, D_OUT),
        _w_spec(3 * D_OUT, D_OUT), _w_spec(1, D_OUT),
    ],
    out_specs=[
        pl.BlockSpec((BM, D_OUT), lambda i: (i, 0)),
        _w_spec(1, D_OUT), _w_spec(1, D_OUT),
    ],
    out_shape=[
        jax.ShapeDtypeStruct((N, D_OUT), jnp.float32),
        jax.ShapeDtypeStruct((1, D_OUT), jnp.float32),
        jax.ShapeDtypeStruct((1, D_OUT), jnp.float32),
    ],
)


def _dense_b_body(o_ref, sum_ref, sq_ref, gamma_ref, beta_ref, out_ref):
    mu = sum_ref[...] * (1.0 / N)
    var = sq_ref[...] * (1.0 / N) - mu * mu
    o = o_ref[...]
    o = (o - mu) / jnp.sqrt(var + 1e-5) * gamma_ref[...] + beta_ref[...]
    out_ref[...] = jnp.maximum(o + o, 0.0)


_dense_b = pl.pallas_call(
    _dense_b_body,
    grid=(GRID,),
    in_specs=[
        pl.BlockSpec((BM, D_OUT), lambda i: (i, 0)),
        _w_spec(1, D_OUT), _w_spec(1, D_OUT),
        _w_spec(1, D_OUT), _w_spec(1, D_OUT),
    ],
    out_specs=pl.BlockSpec((BM, D_OUT), lambda i: (i, 0)),
    out_shape=jax.ShapeDtypeStruct((N, D_OUT), jnp.float32),
)


def kernel(x, edge_index, edge_attr, W_l, b_l, W_r, W_emb, b_emb,
           W_te, b_te, W_ta, b_ta, W_gate, b_gate, gamma, beta):
    row = edge_index[0].astype(jnp.int32).reshape(NS, NCHUNK, CHUNK)
    col = edge_index[1].astype(jnp.int32).reshape(NS, NCHUNK, CHUNK)
    col2 = edge_index[1].astype(jnp.int32).reshape(NS, EVREGS, 16)
    x2 = jnp.stack([x[:, :D_HALF], x[:, D_HALF:]])
    zrow = jnp.zeros((N_ACC, D_HALF), jnp.float32)
    zdeg = jnp.zeros((N_ACC,), jnp.float32)
    neg1 = jnp.full((N_PAD,), -1, jnp.int32)

    agg_p, deg_p, elast = _edge_pass(x2, row, col, col2, zrow, zdeg, neg1)
    mask, attr_sel = _select_pass(elast, edge_attr)

    o, osum, osq = _dense_a(
        x, agg_p[0, :N], agg_p[1, :N],
        deg_p[:N].reshape(N, 1),
        attr_sel[:N], mask[:N].reshape(N, 1),
        W_l.T, b_l.reshape(1, -1), W_r.T,
        W_emb.T, b_emb.reshape(1, -1), W_te.T, b_te.reshape(1, -1),
        W_ta.T, b_ta.reshape(1, -1),
        W_gate.T, b_gate.reshape(1, -1))
    return _dense_b(o, osum, osq, gamma.reshape(1, -1), beta.reshape(1, -1))


# speedup vs baseline: 13.6537x; 1.1869x over previous
import functools

import jax
import jax.numpy as jnp
from jax import lax
from jax.experimental import pallas as pl
from jax.experimental.pallas import tpu as pltpu
from jax.experimental.pallas import tpu_sc as plsc

N = 10000
E = 320000
D_IN = 128
D_OUT = 128
D_EDGE = 16
D_HALF = D_IN // 2

NC = 2
NS = 16
E_PER_TEC = E // NS
CHUNK = 125
NCHUNK = E_PER_TEC // CHUNK
EVREGS = E_PER_TEC // 16
N_ACC = 10240
ROWS_PER_SUB = N_ACC // NS
N_PAD = 10240
NW = NC * NS
SEL_PER_TEC = N_PAD // NW
SEL_VREGS = SEL_PER_TEC // 16
SEL_CHUNK = 80


def _edge_pass_body(x2_hbm, row_hbm, col_hbm, col2_hbm, zrow_hbm, zdeg_hbm,
                    neg1_hbm,
                    agg_out, deg_out, elast_out,
                    row_v, col_v, col2_v, rows_v0, rows_v1, ones_v, elast_v,
                    sem0, sem1, ssem0, ssem1, agg_sh, deg_sh):
    c = lax.axis_index("c")
    s = lax.axis_index("s")

    pltpu.sync_copy(zrow_hbm.at[pl.ds(s * ROWS_PER_SUB, ROWS_PER_SUB)],
                    agg_sh.at[pl.ds(s * ROWS_PER_SUB, ROWS_PER_SUB)])

    @pl.when(s == 0)
    def _():
        pltpu.sync_copy(zdeg_hbm, deg_sh)

    pltpu.sync_copy(row_hbm.at[s], row_v)
    pltpu.sync_copy(col_hbm.at[s], col_v)

    @pl.when(c == 0)
    def _():
        pltpu.sync_copy(col2_hbm.at[s], col2_v)
        pltpu.sync_copy(neg1_hbm, elast_v)

    for j in range(8):
        ones_v[pl.ds(j * 16, 16)] = jnp.full((16,), 1.0, jnp.float32)

    plsc.subcore_barrier()

    lane = lax.iota(jnp.int32, 16)
    e_base = s * E_PER_TEC
    xh = x2_hbm.at[c]
    ones_c = ones_v.at[pl.ds(0, CHUNK)]

    pltpu.async_copy(xh.at[row_v.at[0]], rows_v0, sem0)
    pltpu.async_copy(xh.at[row_v.at[1]], rows_v1, sem1)

    def body(k, carry):
        i0 = 2 * k
        i1 = 2 * k + 1
        pltpu.make_async_copy(xh.at[row_v.at[i0]], rows_v0, sem0).wait()
        pltpu.async_copy(rows_v0, agg_sh.at[col_v.at[i0]], ssem0, add=True)

        @pl.when(c == 1)
        def _():
            pltpu.sync_copy(ones_c, deg_sh.at[col_v.at[i0]], add=True)

        pltpu.make_async_copy(xh.at[row_v.at[i1]], rows_v1, sem1).wait()
        pltpu.async_copy(rows_v1, agg_sh.at[col_v.at[i1]], ssem1, add=True)

        @pl.when(c == 1)
        def _():
            pltpu.sync_copy(ones_c, deg_sh.at[col_v.at[i1]], add=True)

        @pl.when(k < NCHUNK // 2 - 1)
        def _():
            pltpu.make_async_copy(
                rows_v0, agg_sh.at[col_v.at[i0]], ssem0).wait()
            pltpu.async_copy(xh.at[row_v.at[i0 + 2]], rows_v0, sem0)
            pltpu.make_async_copy(
                rows_v1, agg_sh.at[col_v.at[i1]], ssem1).wait()
            pltpu.async_copy(xh.at[row_v.at[i1 + 2]], rows_v1, sem1)

        return carry

    lax.fori_loop(0, NCHUNK // 2, body, 0)

    pltpu.make_async_copy(
        rows_v0, agg_sh.at[col_v.at[NCHUNK - 2]], ssem0).wait()
    pltpu.make_async_copy(
        rows_v1, agg_sh.at[col_v.at[NCHUNK - 1]], ssem1).wait()

    @pl.when(c == 0)
    def _():
        def ebody(v, carry):
            col16 = col2_v[v]
            e16 = (e_base + v * 16) + lane
            plsc.store_scatter(elast_v, [col16], e16)
            return carry

        lax.fori_loop(0, EVREGS, ebody, 0)
        pltpu.sync_copy(elast_v, elast_out.at[s])

    plsc.subcore_barrier()
    pltpu.sync_copy(agg_sh.at[pl.ds(s * ROWS_PER_SUB, ROWS_PER_SUB)],
                    agg_out.at[c, pl.ds(s * ROWS_PER_SUB, ROWS_PER_SUB)])

    @pl.when((s == 0) & (c == 1))
    def _():
        pltpu.sync_copy(deg_sh, deg_out)


_edge_pass = functools.partial(
    pl.kernel,
    mesh=plsc.VectorSubcoreMesh(core_axis_name="c", subcore_axis_name="s"),
    compiler_params=pltpu.CompilerParams(
        needs_layout_passes=False, use_tc_tiling_on_sc=False),
    out_type=[
        jax.ShapeDtypeStruct((NC, N_ACC, D_HALF), jnp.float32),
        jax.ShapeDtypeStruct((N_ACC,), jnp.float32),
        jax.ShapeDtypeStruct((NS, N_PAD), jnp.int32),
    ],
    scratch_types=[
        pltpu.VMEM((NCHUNK, CHUNK), jnp.int32),
        pltpu.VMEM((NCHUNK, CHUNK), jnp.int32),
        pltpu.VMEM((EVREGS, 16), jnp.int32),
        pltpu.VMEM((CHUNK, D_HALF), jnp.float32),
        pltpu.VMEM((CHUNK, D_HALF), jnp.float32),
        pltpu.VMEM((128,), jnp.float32),
        pltpu.VMEM((N_PAD,), jnp.int32),
        pltpu.SemaphoreType.DMA,
        pltpu.SemaphoreType.DMA,
        pltpu.SemaphoreType.DMA,
        pltpu.SemaphoreType.DMA,
        pltpu.VMEM_SHARED((N_ACC, D_HALF), jnp.float32),
        pltpu.VMEM_SHARED((N_ACC,), jnp.float32),
    ],
)(_edge_pass_body)


def _select_pass_body(elast_hbm, attr_hbm,
                      mask_out, attr_out,
                      tbl_v, idx_v, mask_v, arows_v, sem):
    c = lax.axis_index("c")
    s = lax.axis_index("s")
    wid = c * NS + s
    base = wid * SEL_PER_TEC

    pltpu.sync_copy(elast_hbm.at[:, pl.ds(base, SEL_PER_TEC)], tbl_v)

    def combine(k, carry):
        m = tbl_v[0, pl.ds(k * 16, 16)]
        for j in range(1, NS):
            m = jnp.maximum(m, tbl_v[j, pl.ds(k * 16, 16)])
        idx_v[pl.ds(k * 16, 16)] = jnp.maximum(m, 0)
        mask_v[pl.ds(k * 16, 16)] = jnp.where(m >= 0, 1.0, 0.0)
        return carry

    lax.fori_loop(0, SEL_VREGS, combine, 0)

    pltpu.sync_copy(mask_v, mask_out.at[pl.ds(base, SEL_PER_TEC)])

    def gather(t, carry):
        pltpu.async_copy(
            attr_hbm.at[idx_v.at[pl.ds(t * SEL_CHUNK, SEL_CHUNK)]],
            arows_v.at[pl.ds(t * SEL_CHUNK, SEL_CHUNK)], sem).wait()
        return carry

    lax.fori_loop(0, SEL_PER_TEC // SEL_CHUNK, gather, 0)
    pltpu.sync_copy(arows_v, attr_out.at[pl.ds(base, SEL_PER_TEC)])


_select_pass = functools.partial(
    pl.kernel,
    mesh=plsc.VectorSubcoreMesh(core_axis_name="c", subcore_axis_name="s"),
    compiler_params=pltpu.CompilerParams(
        needs_layout_passes=False, use_tc_tiling_on_sc=False),
    out_type=[
        jax.ShapeDtypeStruct((N_PAD,), jnp.float32),
        jax.ShapeDtypeStruct((N_PAD, D_EDGE), jnp.float32),
    ],
    scratch_types=[
        pltpu.VMEM((NS, SEL_PER_TEC), jnp.int32),
        pltpu.VMEM((SEL_PER_TEC,), jnp.int32),
        pltpu.VMEM((SEL_PER_TEC,), jnp.float32),
        pltpu.VMEM((SEL_PER_TEC, D_EDGE), jnp.float32),
        pltpu.SemaphoreType.DMA,
    ],
)(_select_pass_body)


def _mm(a, b):
    return jax.lax.dot_general(
        a, b, (((1,), (0,)), ((), ())),
        precision=jax.lax.Precision.DEFAULT,
        preferred_element_type=jnp.float32)


BM = 2000
GRID = N // BM


def _dense_a_body(x_ref, agg0_ref, agg1_ref, deg_ref,
                  attr_ref, mask_ref, WlT_ref, bl_ref, WrT_ref,
                  WembT_ref, bemb_ref, WteT_ref, bte_ref, WtaT_ref, bta_ref,
                  WgT_ref, bg_ref, o_ref, sum_ref, sq_ref):
    x = x_ref[...]
    deg = jnp.maximum(deg_ref[...], 1.0)
    h0 = agg0_ref[...] / deg
    h1 = agg1_ref[...] / deg
    WlT = WlT_ref[...]
    out = (_mm(h0, WlT[:D_HALF]) + _mm(h1, WlT[D_HALF:]) + bl_ref[...]
           + _mm(x, WrT_ref[...]))

    attr = attr_ref[...]
    emb = jnp.maximum(_mm(attr, WembT_ref[...]) + bemb_ref[...], 0.0)
    t_emb = _mm(emb, WteT_ref[...]) + bte_ref[...]
    t_attr = _mm(attr, WtaT_ref[...]) + bta_ref[...]
    t = t_emb + t_attr

    WgT = WgT_ref[...]
    logits = (_mm(out, WgT[:D_OUT]) + _mm(t, WgT[D_OUT:2 * D_OUT])
              + _mm(t_attr, WgT[2 * D_OUT:]) + bg_ref[...])
    gate = 1.0 / (1.0 + jnp.exp(-logits))
    contrib = gate * t * mask_ref[...]

    o = out + contrib
    o_ref[...] = o

    @pl.when(pl.program_id(0) == 0)
    def _():
        sum_ref[...] = jnp.zeros_like(sum_ref)
        sq_ref[...] = jnp.zeros_like(sq_ref)

    sum_ref[...] += jnp.sum(o, axis=0, keepdims=True)
    sq_ref[...] += jnp.sum(o * o, axis=0, keepdims=True)


_row_spec = pl.BlockSpec((BM, D_IN), lambda i: (i, 0))
_half_spec = pl.BlockSpec((BM, D_HALF), lambda i: (i, 0))
_col_spec = pl.BlockSpec((BM, 1), lambda i: (i, 0))


def _w_spec(r, c):
    return pl.BlockSpec((r, c), lambda i: (0, 0))


_dense_a = pl.pallas_call(
    _dense_a_body,
    grid=(GRID,),
    in_specs=[
        _row_spec, _half_spec, _half_spec, _col_spec,
        pl.BlockSpec((BM, D_EDGE), lambda i: (i, 0)), _col_spec,
        _w_spec(D_IN, D_OUT), _w_spec(1, D_OUT), _w_spec(D_IN, D_OUT),
        _w_spec(D_EDGE, D_EDGE), _w_spec(1, D_EDGE),
        _w_spec(D_EDGE, D_OUT), _w_spec(1, D_OUT),
        _w_spec(D_EDGE, D_OUT), _w_spec(1, D_OUT),
        _w_spec(3 * D_OUT, D_OUT), _w_spec(1, D_OUT),
    ],
    out_specs=[
        pl.BlockSpec((BM, D_OUT), lambda i: (i, 0)),
        _w_spec(1, D_OUT), _w_spec(1, D_OUT),
    ],
    out_shape=[
        jax.ShapeDtypeStruct((N, D_OUT), jnp.float32),
        jax.ShapeDtypeStruct((1, D_OUT), jnp.float32),
        jax.ShapeDtypeStruct((1, D_OUT), jnp.float32),
    ],
)


def _dense_b_body(o_ref, sum_ref, sq_ref, gamma_ref, beta_ref, out_ref):
    mu = sum_ref[...] * (1.0 / N)
    var = sq_ref[...] * (1.0 / N) - mu * mu
    o = o_ref[...]
    o = (o - mu) / jnp.sqrt(var + 1e-5) * gamma_ref[...] + beta_ref[...]
    out_ref[...] = jnp.maximum(o + o, 0.0)


_dense_b = pl.pallas_call(
    _dense_b_body,
    grid=(GRID,),
    in_specs=[
        pl.BlockSpec((BM, D_OUT), lambda i: (i, 0)),
        _w_spec(1, D_OUT), _w_spec(1, D_OUT),
        _w_spec(1, D_OUT), _w_spec(1, D_OUT),
    ],
    out_specs=pl.BlockSpec((BM, D_OUT), lambda i: (i, 0)),
    out_shape=jax.ShapeDtypeStruct((N, D_OUT), jnp.float32),
)


def kernel(x, edge_index, edge_attr, W_l, b_l, W_r, W_emb, b_emb,
           W_te, b_te, W_ta, b_ta, W_gate, b_gate, gamma, beta):
    row = edge_index[0].astype(jnp.int32).reshape(NS, NCHUNK, CHUNK)
    col = edge_index[1].astype(jnp.int32).reshape(NS, NCHUNK, CHUNK)
    col2 = edge_index[1].astype(jnp.int32).reshape(NS, EVREGS, 16)
    x2 = jnp.stack([x[:, :D_HALF], x[:, D_HALF:]])
    zrow = jnp.zeros((N_ACC, D_HALF), jnp.float32)
    zdeg = jnp.zeros((N_ACC,), jnp.float32)
    neg1 = jnp.full((N_PAD,), -1, jnp.int32)

    agg_p, deg_p, elast = _edge_pass(x2, row, col, col2, zrow, zdeg, neg1)
    mask, attr_sel = _select_pass(elast, edge_attr)

    o, osum, osq = _dense_a(
        x, agg_p[0, :N], agg_p[1, :N],
        deg_p[:N].reshape(N, 1),
        attr_sel[:N], mask[:N].reshape(N, 1),
        W_l.T, b_l.reshape(1, -1), W_r.T,
        W_emb.T, b_emb.reshape(1, -1), W_te.T, b_te.reshape(1, -1),
        W_ta.T, b_ta.reshape(1, -1),
        W_gate.T, b_gate.reshape(1, -1))
    return _dense_b(o, osum, osq, gamma.reshape(1, -1), beta.reshape(1, -1))


# speedup vs baseline: 14.0881x; 1.0318x over previous
import functools

import jax
import jax.numpy as jnp
from jax import lax
from jax.experimental import pallas as pl
from jax.experimental.pallas import tpu as pltpu
from jax.experimental.pallas import tpu_sc as plsc

N = 10000
E = 320000
D_IN = 128
D_OUT = 128
D_EDGE = 16
D_HALF = D_IN // 2

NC = 2
NS = 16
E_PER_TEC = E // NS
CHUNK = 125
NCHUNK = E_PER_TEC // CHUNK
EVREGS = E_PER_TEC // 16
N_ACC = 10240
ROWS_PER_SUB = N_ACC // NS
N_PAD = 10240
NW = NC * NS
SEL_PER_TEC = N_PAD // NW
SEL_VREGS = SEL_PER_TEC // 16
SEL_CHUNK = 80


def _edge_pass_body(x2_hbm, ei_hbm, col2_hbm, zrow_hbm, zdeg_hbm,
                    neg1_hbm,
                    agg_out, deg_out, elast_out,
                    row_v, col_v, col2_v, rows_v0, rows_v1, ones_v, elast_v,
                    sem0, sem1, ssem0, ssem1, agg_sh, deg_sh):
    c = lax.axis_index("c")
    s = lax.axis_index("s")

    pltpu.sync_copy(zrow_hbm.at[pl.ds(s * ROWS_PER_SUB, ROWS_PER_SUB)],
                    agg_sh.at[pl.ds(s * ROWS_PER_SUB, ROWS_PER_SUB)])

    @pl.when(s == 0)
    def _():
        pltpu.sync_copy(zdeg_hbm, deg_sh)

    pltpu.sync_copy(ei_hbm.at[0, s], row_v)
    pltpu.sync_copy(ei_hbm.at[1, s], col_v)

    @pl.when(c == 0)
    def _():
        pltpu.sync_copy(col2_hbm.at[s], col2_v)
        pltpu.sync_copy(neg1_hbm, elast_v)

    for j in range(8):
        ones_v[pl.ds(j * 16, 16)] = jnp.full((16,), 1.0, jnp.float32)

    plsc.subcore_barrier()

    lane = lax.iota(jnp.int32, 16)
    e_base = s * E_PER_TEC
    xh = x2_hbm.at[c]
    ones_c = ones_v.at[pl.ds(0, CHUNK)]

    pltpu.async_copy(xh.at[row_v.at[0]], rows_v0, sem0)
    pltpu.async_copy(xh.at[row_v.at[1]], rows_v1, sem1)

    def body(k, carry):
        i0 = 2 * k
        i1 = 2 * k + 1
        pltpu.make_async_copy(xh.at[row_v.at[i0]], rows_v0, sem0).wait()
        pltpu.async_copy(rows_v0, agg_sh.at[col_v.at[i0]], ssem0, add=True)

        @pl.when(c == 1)
        def _():
            pltpu.sync_copy(ones_c, deg_sh.at[col_v.at[i0]], add=True)

        pltpu.make_async_copy(xh.at[row_v.at[i1]], rows_v1, sem1).wait()
        pltpu.async_copy(rows_v1, agg_sh.at[col_v.at[i1]], ssem1, add=True)

        @pl.when(c == 1)
        def _():
            pltpu.sync_copy(ones_c, deg_sh.at[col_v.at[i1]], add=True)

        @pl.when(k < NCHUNK // 2 - 1)
        def _():
            pltpu.make_async_copy(
                rows_v0, agg_sh.at[col_v.at[i0]], ssem0).wait()
            pltpu.async_copy(xh.at[row_v.at[i0 + 2]], rows_v0, sem0)
            pltpu.make_async_copy(
                rows_v1, agg_sh.at[col_v.at[i1]], ssem1).wait()
            pltpu.async_copy(xh.at[row_v.at[i1 + 2]], rows_v1, sem1)

        return carry

    lax.fori_loop(0, NCHUNK // 2, body, 0)

    pltpu.make_async_copy(
        rows_v0, agg_sh.at[col_v.at[NCHUNK - 2]], ssem0).wait()
    pltpu.make_async_copy(
        rows_v1, agg_sh.at[col_v.at[NCHUNK - 1]], ssem1).wait()

    @pl.when(c == 0)
    def _():
        def ebody(v, carry):
            col16 = col2_v[v]
            e16 = (e_base + v * 16) + lane
            plsc.store_scatter(elast_v, [col16], e16)
            return carry

        lax.fori_loop(0, EVREGS, ebody, 0)
        pltpu.sync_copy(elast_v, elast_out.at[s])

    plsc.subcore_barrier()
    pltpu.sync_copy(agg_sh.at[pl.ds(s * ROWS_PER_SUB, ROWS_PER_SUB)],
                    agg_out.at[c, pl.ds(s * ROWS_PER_SUB, ROWS_PER_SUB)])

    @pl.when((s == 0) & (c == 1))
    def _():
        pltpu.sync_copy(deg_sh, deg_out)


_edge_pass = functools.partial(
    pl.kernel,
    mesh=plsc.VectorSubcoreMesh(core_axis_name="c", subcore_axis_name="s"),
    compiler_params=pltpu.CompilerParams(
        needs_layout_passes=False, use_tc_tiling_on_sc=False),
    out_type=[
        jax.ShapeDtypeStruct((NC, N_ACC, D_HALF), jnp.float32),
        jax.ShapeDtypeStruct((N_ACC,), jnp.float32),
        jax.ShapeDtypeStruct((NS, N_PAD), jnp.int32),
    ],
    scratch_types=[
        pltpu.VMEM((NCHUNK, CHUNK), jnp.int32),
        pltpu.VMEM((NCHUNK, CHUNK), jnp.int32),
        pltpu.VMEM((EVREGS, 16), jnp.int32),
        pltpu.VMEM((CHUNK, D_HALF), jnp.float32),
        pltpu.VMEM((CHUNK, D_HALF), jnp.float32),
        pltpu.VMEM((128,), jnp.float32),
        pltpu.VMEM((N_PAD,), jnp.int32),
        pltpu.SemaphoreType.DMA,
        pltpu.SemaphoreType.DMA,
        pltpu.SemaphoreType.DMA,
        pltpu.SemaphoreType.DMA,
        pltpu.VMEM_SHARED((N_ACC, D_HALF), jnp.float32),
        pltpu.VMEM_SHARED((N_ACC,), jnp.float32),
    ],
)(_edge_pass_body)


def _select_pass_body(elast_hbm, attr_hbm,
                      mask_out, attr_out,
                      tbl_v, idx_v, mask_v, arows_v, sem):
    c = lax.axis_index("c")
    s = lax.axis_index("s")
    wid = c * NS + s
    base = wid * SEL_PER_TEC

    pltpu.sync_copy(elast_hbm.at[:, pl.ds(base, SEL_PER_TEC)], tbl_v)

    def combine(k, carry):
        m = tbl_v[0, pl.ds(k * 16, 16)]
        for j in range(1, NS):
            m = jnp.maximum(m, tbl_v[j, pl.ds(k * 16, 16)])
        idx_v[pl.ds(k * 16, 16)] = jnp.maximum(m, 0)
        mask_v[pl.ds(k * 16, 16)] = jnp.where(m >= 0, 1.0, 0.0)
        return carry

    lax.fori_loop(0, SEL_VREGS, combine, 0)

    pltpu.sync_copy(mask_v, mask_out.at[pl.ds(base, SEL_PER_TEC)])

    def gather(t, carry):
        pltpu.async_copy(
            attr_hbm.at[idx_v.at[pl.ds(t * SEL_CHUNK, SEL_CHUNK)]],
            arows_v.at[pl.ds(t * SEL_CHUNK, SEL_CHUNK)], sem).wait()
        return carry

    lax.fori_loop(0, SEL_PER_TEC // SEL_CHUNK, gather, 0)
    pltpu.sync_copy(arows_v, attr_out.at[pl.ds(base, SEL_PER_TEC)])


_select_pass = functools.partial(
    pl.kernel,
    mesh=plsc.VectorSubcoreMesh(core_axis_name="c", subcore_axis_name="s"),
    compiler_params=pltpu.CompilerParams(
        needs_layout_passes=False, use_tc_tiling_on_sc=False),
    out_type=[
        jax.ShapeDtypeStruct((N_PAD,), jnp.float32),
        jax.ShapeDtypeStruct((N_PAD, D_EDGE), jnp.float32),
    ],
    scratch_types=[
        pltpu.VMEM((NS, SEL_PER_TEC), jnp.int32),
        pltpu.VMEM((SEL_PER_TEC,), jnp.int32),
        pltpu.VMEM((SEL_PER_TEC,), jnp.float32),
        pltpu.VMEM((SEL_PER_TEC, D_EDGE), jnp.float32),
        pltpu.SemaphoreType.DMA,
    ],
)(_select_pass_body)


def _mm(a, b):
    return jax.lax.dot_general(
        a, b, (((1,), (0,)), ((), ())),
        precision=jax.lax.Precision.DEFAULT,
        preferred_element_type=jnp.float32)


BM = 2000
GRID = N // BM


def _dense_a_body(x_ref, agg0_ref, agg1_ref, deg_ref,
                  attr_ref, mask_ref, WlT_ref, bl_ref, WrT_ref,
                  WembT_ref, bemb_ref, WteT_ref, bte_ref, WtaT_ref, bta_ref,
                  WgT_ref, bg_ref, o_ref, sum_ref, sq_ref):
    x = x_ref[...]
    deg = jnp.maximum(deg_ref[...], 1.0)
    h0 = agg0_ref[...] / deg
    h1 = agg1_ref[...] / deg
    WlT = WlT_ref[...]
    out = (_mm(h0, WlT[:D_HALF]) + _mm(h1, WlT[D_HALF:]) + bl_ref[...]
           + _mm(x, WrT_ref[...]))

    attr = attr_ref[...]
    emb = jnp.maximum(_mm(attr, WembT_ref[...]) + bemb_ref[...], 0.0)
    t_emb = _mm(emb, WteT_ref[...]) + bte_ref[...]
    t_attr = _mm(attr, WtaT_ref[...]) + bta_ref[...]
    t = t_emb + t_attr

    WgT = WgT_ref[...]
    logits = (_mm(out, WgT[:D_OUT]) + _mm(t, WgT[D_OUT:2 * D_OUT])
              + _mm(t_attr, WgT[2 * D_OUT:]) + bg_ref[...])
    gate = 1.0 / (1.0 + jnp.exp(-logits))
    contrib = gate * t * mask_ref[...]

    o = out + contrib
    o_ref[...] = o

    @pl.when(pl.program_id(0) == 0)
    def _():
        sum_ref[...] = jnp.zeros_like(sum_ref)
        sq_ref[...] = jnp.zeros_like(sq_ref)

    sum_ref[...] += jnp.sum(o, axis=0, keepdims=True)
    sq_ref[...] += jnp.sum(o * o, axis=0, keepdims=True)


_row_spec = pl.BlockSpec((BM, D_IN), lambda i: (i, 0))
_half_spec = pl.BlockSpec((BM, D_HALF), lambda i: (i, 0))
_col_spec = pl.BlockSpec((BM, 1), lambda i: (i, 0))


def _w_spec(r, c):
    return pl.BlockSpec((r, c), lambda i: (0, 0))


_dense_a = pl.pallas_call(
    _dense_a_body,
    grid=(GRID,),
    in_specs=[
        _row_spec, _half_spec, _half_spec, _col_spec,
        pl.BlockSpec((BM, D_EDGE), lambda i: (i, 0)), _col_spec,
        _w_spec(D_IN, D_OUT), _w_spec(1, D_OUT), _w_spec(D_IN, D_OUT),
        _w_spec(D_EDGE, D_EDGE), _w_spec(1, D_EDGE),
        _w_spec(D_EDGE, D_OUT), _w_spec(1, D_OUT),
        _w_spec(D_EDGE, D_OUT), _w_spec(1, D_OUT),
        _w_spec(3 * D_OUT, D_OUT), _w_spec(1, D_OUT),
    ],
    out_specs=[
        pl.BlockSpec((BM, D_OUT), lambda i: (i, 0)),
        _w_spec(1, D_OUT), _w_spec(1, D_OUT),
    ],
    out_shape=[
        jax.ShapeDtypeStruct((N, D_OUT), jnp.float32),
        jax.ShapeDtypeStruct((1, D_OUT), jnp.float32),
        jax.ShapeDtypeStruct((1, D_OUT), jnp.float32),
    ],
)


def _dense_b_body(o_ref, sum_ref, sq_ref, gamma_ref, beta_ref, out_ref):
    mu = sum_ref[...] * (1.0 / N)
    var = sq_ref[...] * (1.0 / N) - mu * mu
    o = o_ref[...]
    o = (o - mu) / jnp.sqrt(var + 1e-5) * gamma_ref[...] + beta_ref[...]
    out_ref[...] = jnp.maximum(o + o, 0.0)


_dense_b = pl.pallas_call(
    _dense_b_body,
    grid=(GRID,),
    in_specs=[
        pl.BlockSpec((BM, D_OUT), lambda i: (i, 0)),
        _w_spec(1, D_OUT), _w_spec(1, D_OUT),
        _w_spec(1, D_OUT), _w_spec(1, D_OUT),
    ],
    out_specs=pl.BlockSpec((BM, D_OUT), lambda i: (i, 0)),
    out_shape=jax.ShapeDtypeStruct((N, D_OUT), jnp.float32),
)


def kernel(x, edge_index, edge_attr, W_l, b_l, W_r, W_emb, b_emb,
           W_te, b_te, W_ta, b_ta, W_gate, b_gate, gamma, beta):
    ei = edge_index.astype(jnp.int32).reshape(2, NS, NCHUNK, CHUNK)
    col2 = ei[1].reshape(NS, EVREGS, 16)
    x2 = jnp.stack([x[:, :D_HALF], x[:, D_HALF:]])
    zrow = jnp.zeros((N_ACC, D_HALF), jnp.float32)
    zdeg = jnp.zeros((N_ACC,), jnp.float32)
    neg1 = jnp.full((N_PAD,), -1, jnp.int32)

    agg_p, deg_p, elast = _edge_pass(x2, ei, col2, zrow, zdeg, neg1)
    mask, attr_sel = _select_pass(elast, edge_attr)

    o, osum, osq = _dense_a(
        x, agg_p[0, :N], agg_p[1, :N],
        deg_p[:N].reshape(N, 1),
        attr_sel[:N], mask[:N].reshape(N, 1),
        W_l.T, b_l.reshape(1, -1), W_r.T,
        W_emb.T, b_emb.reshape(1, -1), W_te.T, b_te.reshape(1, -1),
        W_ta.T, b_ta.reshape(1, -1),
        W_gate.T, b_gate.reshape(1, -1))
    return _dense_b(o, osum, osq, gamma.reshape(1, -1), beta.reshape(1, -1))


# speedup vs baseline: 15.5878x; 1.1064x over previous
import functools

import jax
import jax.numpy as jnp
from jax import lax
from jax.experimental import pallas as pl
from jax.experimental.pallas import tpu as pltpu
from jax.experimental.pallas import tpu_sc as plsc

N = 10000
E = 320000
D_IN = 128
D_OUT = 128
D_EDGE = 16
D_HALF = D_IN // 2

NC = 2
NS = 16
E_PER_TEC = E // NS
CHUNK = 125
NCHUNK = E_PER_TEC // CHUNK
EVREGS = E_PER_TEC // 16
N_ACC = 10240
ROWS_PER_SUB = N_ACC // NS
N_PAD = 10240
NW = NC * NS
SEL_PER_TEC = N_PAD // NW
SEL_VREGS = SEL_PER_TEC // 16
SEL_CHUNK = 80


def _edge_pass_body(x2_hbm, ei_hbm, zrow_hbm, zdeg_hbm,
                    neg1_hbm,
                    agg_out, deg_out, elast_out,
                    row_v, col_v, rows_v0, rows_v1, rows_v2, rows_v3,
                    ones_v, elast_v,
                    sem0, sem1, sem2, sem3, ssem0, ssem1, ssem2, ssem3,
                    agg_sh, deg_sh):
    c = lax.axis_index("c")
    s = lax.axis_index("s")

    pltpu.sync_copy(zrow_hbm.at[pl.ds(s * ROWS_PER_SUB, ROWS_PER_SUB)],
                    agg_sh.at[pl.ds(s * ROWS_PER_SUB, ROWS_PER_SUB)])

    @pl.when(s == 0)
    def _():
        pltpu.sync_copy(zdeg_hbm, deg_sh)

    pltpu.sync_copy(ei_hbm.at[0, s], row_v)
    pltpu.sync_copy(ei_hbm.at[1, s], col_v)

    @pl.when(c == 0)
    def _():
        pltpu.sync_copy(neg1_hbm, elast_v)

    for j in range(8):
        ones_v[pl.ds(j * 16, 16)] = jnp.full((16,), 1.0, jnp.float32)

    plsc.subcore_barrier()

    lane = lax.iota(jnp.int32, 16)
    e_base = s * E_PER_TEC
    xh = x2_hbm.at[c]
    ones_c = ones_v.at[pl.ds(0, CHUNK)]

    bufs = (rows_v0, rows_v1, rows_v2, rows_v3)
    gsems = (sem0, sem1, sem2, sem3)
    ssems = (ssem0, ssem1, ssem2, ssem3)
    NB = 4
    for b in range(NB):
        pltpu.async_copy(xh.at[row_v.at[b]], bufs[b], gsems[b])

    def body(k, carry):
        base = NB * k
        for b in range(NB):
            i = base + b
            pltpu.make_async_copy(xh.at[row_v.at[i]], bufs[b], gsems[b]).wait()
            pltpu.async_copy(bufs[b], agg_sh.at[col_v.at[i]], ssems[b],
                             add=True)

            @pl.when(c == 1)
            def _():
                pltpu.sync_copy(ones_c, deg_sh.at[col_v.at[i]], add=True)

        @pl.when(k < NCHUNK // NB - 1)
        def _():
            for b in range(NB):
                i = base + b
                pltpu.make_async_copy(
                    bufs[b], agg_sh.at[col_v.at[i]], ssems[b]).wait()
                pltpu.async_copy(xh.at[row_v.at[i + NB]], bufs[b], gsems[b])

        return carry

    lax.fori_loop(0, NCHUNK // NB, body, 0)

    for b in range(NB):
        pltpu.make_async_copy(
            bufs[b], agg_sh.at[col_v.at[NCHUNK - NB + b]], ssems[b]).wait()

    @pl.when(c == 0)
    def _():
        def ebody(v, carry):
            p = v * 16 + lane
            col16 = plsc.load_gather(col_v, [p // CHUNK, p % CHUNK])
            e16 = e_base + p
            plsc.store_scatter(elast_v, [col16], e16)
            return carry

        lax.fori_loop(0, EVREGS, ebody, 0)
        pltpu.sync_copy(elast_v, elast_out.at[s])

    plsc.subcore_barrier()
    pltpu.sync_copy(agg_sh.at[pl.ds(s * ROWS_PER_SUB, ROWS_PER_SUB)],
                    agg_out.at[c, pl.ds(s * ROWS_PER_SUB, ROWS_PER_SUB)])

    @pl.when((s == 0) & (c == 1))
    def _():
        pltpu.sync_copy(deg_sh, deg_out)


_edge_pass = functools.partial(
    pl.kernel,
    mesh=plsc.VectorSubcoreMesh(core_axis_name="c", subcore_axis_name="s"),
    compiler_params=pltpu.CompilerParams(
        needs_layout_passes=False, use_tc_tiling_on_sc=False),
    out_type=[
        jax.ShapeDtypeStruct((NC, N_ACC, D_HALF), jnp.float32),
        jax.ShapeDtypeStruct((N_ACC,), jnp.float32),
        jax.ShapeDtypeStruct((NS, N_PAD), jnp.int32),
    ],
    scratch_types=[
        pltpu.VMEM((NCHUNK, CHUNK), jnp.int32),
        pltpu.VMEM((NCHUNK, CHUNK), jnp.int32),
        pltpu.VMEM((CHUNK, D_HALF), jnp.float32),
        pltpu.VMEM((CHUNK, D_HALF), jnp.float32),
        pltpu.VMEM((CHUNK, D_HALF), jnp.float32),
        pltpu.VMEM((CHUNK, D_HALF), jnp.float32),
        pltpu.VMEM((128,), jnp.float32),
        pltpu.VMEM((N_PAD,), jnp.int32),
        pltpu.SemaphoreType.DMA,
        pltpu.SemaphoreType.DMA,
        pltpu.SemaphoreType.DMA,
        pltpu.SemaphoreType.DMA,
        pltpu.SemaphoreType.DMA,
        pltpu.SemaphoreType.DMA,
        pltpu.SemaphoreType.DMA,
        pltpu.SemaphoreType.DMA,
        pltpu.VMEM_SHARED((N_ACC, D_HALF), jnp.float32),
        pltpu.VMEM_SHARED((N_ACC,), jnp.float32),
    ],
)(_edge_pass_body)


def _select_pass_body(elast_hbm, attr_hbm,
                      mask_out, attr_out,
                      tbl_v, idx_v, mask_v, arows_v, sem):
    c = lax.axis_index("c")
    s = lax.axis_index("s")
    wid = c * NS + s
    base = wid * SEL_PER_TEC

    pltpu.sync_copy(elast_hbm.at[:, pl.ds(base, SEL_PER_TEC)], tbl_v)

    def combine(k, carry):
        m = tbl_v[0, pl.ds(k * 16, 16)]
        for j in range(1, NS):
            m = jnp.maximum(m, tbl_v[j, pl.ds(k * 16, 16)])
        idx_v[pl.ds(k * 16, 16)] = jnp.maximum(m, 0)
        mask_v[pl.ds(k * 16, 16)] = jnp.where(m >= 0, 1.0, 0.0)
        return carry

    lax.fori_loop(0, SEL_VREGS, combine, 0)

    pltpu.sync_copy(mask_v, mask_out.at[pl.ds(base, SEL_PER_TEC)])

    def gather(t, carry):
        pltpu.async_copy(
            attr_hbm.at[idx_v.at[pl.ds(t * SEL_CHUNK, SEL_CHUNK)]],
            arows_v.at[pl.ds(t * SEL_CHUNK, SEL_CHUNK)], sem).wait()
        return carry

    lax.fori_loop(0, SEL_PER_TEC // SEL_CHUNK, gather, 0)
    pltpu.sync_copy(arows_v, attr_out.at[pl.ds(base, SEL_PER_TEC)])


_select_pass = functools.partial(
    pl.kernel,
    mesh=plsc.VectorSubcoreMesh(core_axis_name="c", subcore_axis_name="s"),
    compiler_params=pltpu.CompilerParams(
        needs_layout_passes=False, use_tc_tiling_on_sc=False),
    out_type=[
        jax.ShapeDtypeStruct((N_PAD,), jnp.float32),
        jax.ShapeDtypeStruct((N_PAD, D_EDGE), jnp.float32),
    ],
    scratch_types=[
        pltpu.VMEM((NS, SEL_PER_TEC), jnp.int32),
        pltpu.VMEM((SEL_PER_TEC,), jnp.int32),
        pltpu.VMEM((SEL_PER_TEC,), jnp.float32),
        pltpu.VMEM((SEL_PER_TEC, D_EDGE), jnp.float32),
        pltpu.SemaphoreType.DMA,
    ],
)(_select_pass_body)


def _mm(a, b):
    return jax.lax.dot_general(
        a, b, (((1,), (0,)), ((), ())),
        precision=jax.lax.Precision.DEFAULT,
        preferred_element_type=jnp.float32)


BM = 2000
GRID = N // BM


def _dense_a_body(x_ref, agg0_ref, agg1_ref, deg_ref,
                  attr_ref, mask_ref, WlT_ref, bl_ref, WrT_ref,
                  WembT_ref, bemb_ref, WteT_ref, bte_ref, WtaT_ref, bta_ref,
                  WgT_ref, bg_ref, o_ref, sum_ref, sq_ref):
    x = x_ref[...]
    deg = jnp.maximum(deg_ref[...], 1.0)
    h0 = agg0_ref[...] / deg
    h1 = agg1_ref[...] / deg
    WlT = WlT_ref[...]
    out = (_mm(h0, WlT[:D_HALF]) + _mm(h1, WlT[D_HALF:]) + bl_ref[...]
           + _mm(x, WrT_ref[...]))

    attr = attr_ref[...]
    emb = jnp.maximum(_mm(attr, WembT_ref[...]) + bemb_ref[...], 0.0)
    t_emb = _mm(emb, WteT_ref[...]) + bte_ref[...]
    t_attr = _mm(attr, WtaT_ref[...]) + bta_ref[...]
    t = t_emb + t_attr

    WgT = WgT_ref[...]
    logits = (_mm(out, WgT[:D_OUT]) + _mm(t, WgT[D_OUT:2 * D_OUT])
              + _mm(t_attr, WgT[2 * D_OUT:]) + bg_ref[...])
    gate = 1.0 / (1.0 + jnp.exp(-logits))
    contrib = gate * t * mask_ref[...]

    o = out + contrib
    o_ref[...] = o

    @pl.when(pl.program_id(0) == 0)
    def _():
        sum_ref[...] = jnp.zeros_like(sum_ref)
        sq_ref[...] = jnp.zeros_like(sq_ref)

    sum_ref[...] += jnp.sum(o, axis=0, keepdims=True)
    sq_ref[...] += jnp.sum(o * o, axis=0, keepdims=True)


_row_spec = pl.BlockSpec((BM, D_IN), lambda i: (i, 0))
_half_spec = pl.BlockSpec((BM, D_HALF), lambda i: (i, 0))
_col_spec = pl.BlockSpec((BM, 1), lambda i: (i, 0))


def _w_spec(r, c):
    return pl.BlockSpec((r, c), lambda i: (0, 0))


_dense_a = pl.pallas_call(
    _dense_a_body,
    grid=(GRID,),
    in_specs=[
        _row_spec, _half_spec, _half_spec, _col_spec,
        pl.BlockSpec((BM, D_EDGE), lambda i: (i, 0)), _col_spec,
        _w_spec(D_IN, D_OUT), _w_spec(1, D_OUT), _w_spec(D_IN, D_OUT),
        _w_spec(D_EDGE, D_EDGE), _w_spec(1, D_EDGE),
        _w_spec(D_EDGE, D_OUT), _w_spec(1, D_OUT),
        _w_spec(D_EDGE, D_OUT), _w_spec(1, D_OUT),
        _w_spec(3 * D_OUT, D_OUT), _w_spec(1, D_OUT),
    ],
    out_specs=[
        pl.BlockSpec((BM, D_OUT), lambda i: (i, 0)),
        _w_spec(1, D_OUT), _w_spec(1, D_OUT),
    ],
    out_shape=[
        jax.ShapeDtypeStruct((N, D_OUT), jnp.float32),
        jax.ShapeDtypeStruct((1, D_OUT), jnp.float32),
        jax.ShapeDtypeStruct((1, D_OUT), jnp.float32),
    ],
)


def _dense_b_body(o_ref, sum_ref, sq_ref, gamma_ref, beta_ref, out_ref):
    mu = sum_ref[...] * (1.0 / N)
    var = sq_ref[...] * (1.0 / N) - mu * mu
    o = o_ref[...]
    o = (o - mu) / jnp.sqrt(var + 1e-5) * gamma_ref[...] + beta_ref[...]
    out_ref[...] = jnp.maximum(o + o, 0.0)


_dense_b = pl.pallas_call(
    _dense_b_body,
    grid=(GRID,),
    in_specs=[
        pl.BlockSpec((BM, D_OUT), lambda i: (i, 0)),
        _w_spec(1, D_OUT), _w_spec(1, D_OUT),
        _w_spec(1, D_OUT), _w_spec(1, D_OUT),
    ],
    out_specs=pl.BlockSpec((BM, D_OUT), lambda i: (i, 0)),
    out_shape=jax.ShapeDtypeStruct((N, D_OUT), jnp.float32),
)


def kernel(x, edge_index, edge_attr, W_l, b_l, W_r, W_emb, b_emb,
           W_te, b_te, W_ta, b_ta, W_gate, b_gate, gamma, beta):
    ei = edge_index.astype(jnp.int32).reshape(2, NS, NCHUNK, CHUNK)
    x2 = jnp.stack([x[:, :D_HALF], x[:, D_HALF:]])
    zrow = jnp.zeros((N_ACC, D_HALF), jnp.float32)
    zdeg = jnp.zeros((N_ACC,), jnp.float32)
    neg1 = jnp.full((N_PAD,), -1, jnp.int32)

    agg_p, deg_p, elast = _edge_pass(x2, ei, zrow, zdeg, neg1)
    mask, attr_sel = _select_pass(elast, edge_attr)

    o, osum, osq = _dense_a(
        x, agg_p[0, :N], agg_p[1, :N],
        deg_p[:N].reshape(N, 1),
        attr_sel[:N], mask[:N].reshape(N, 1),
        W_l.T, b_l.reshape(1, -1), W_r.T,
        W_emb.T, b_emb.reshape(1, -1), W_te.T, b_te.reshape(1, -1),
        W_ta.T, b_ta.reshape(1, -1),
        W_gate.T, b_gate.reshape(1, -1))
    return _dense_b(o, osum, osq, gamma.reshape(1, -1), beta.reshape(1, -1))


# speedup vs baseline: 15.9442x; 1.0229x over previous
import functools

import jax
import jax.numpy as jnp
from jax import lax
from jax.experimental import pallas as pl
from jax.experimental.pallas import tpu as pltpu
from jax.experimental.pallas import tpu_sc as plsc

N = 10000
E = 320000
D_IN = 128
D_OUT = 128
D_EDGE = 16
D_HALF = D_IN // 2

NC = 2
NS = 16
E_PER_TEC = E // NS
CHUNK = 125
NCHUNK = E_PER_TEC // CHUNK
EVREGS = E_PER_TEC // 16
N_ACC = 10240
ROWS_PER_SUB = N_ACC // NS
N_PAD = 10240
NW = NC * NS
SEL_PER_TEC = N_PAD // NW
SEL_VREGS = SEL_PER_TEC // 16
SEL_CHUNK = 80


def _edge_pass_body(x2_hbm, ei_hbm, zrow_hbm, zdeg_hbm,
                    neg1_hbm,
                    agg_out, deg_out, elast_out,
                    row_v, col_v, rows_v0, rows_v1, rows_v2, rows_v3,
                    ones_v, elast_v,
                    sem0, sem1, sem2, sem3, ssem0, ssem1, ssem2, ssem3,
                    agg_sh, deg_sh):
    c = lax.axis_index("c")
    s = lax.axis_index("s")

    pltpu.sync_copy(zrow_hbm.at[pl.ds(s * ROWS_PER_SUB, ROWS_PER_SUB)],
                    agg_sh.at[pl.ds(s * ROWS_PER_SUB, ROWS_PER_SUB)])

    @pl.when(s == 0)
    def _():
        pltpu.sync_copy(zdeg_hbm, deg_sh)

    pltpu.sync_copy(ei_hbm.at[0, s], row_v)
    pltpu.sync_copy(ei_hbm.at[1, s], col_v)

    pltpu.sync_copy(neg1_hbm, elast_v)

    for j in range(8):
        ones_v[pl.ds(j * 16, 16)] = jnp.full((16,), 1.0, jnp.float32)

    plsc.subcore_barrier()

    lane = lax.iota(jnp.int32, 16)
    e_base = s * E_PER_TEC
    xh = x2_hbm.at[c]
    ones_c = ones_v.at[pl.ds(0, CHUNK)]

    bufs = (rows_v0, rows_v1, rows_v2, rows_v3)
    gsems = (sem0, sem1, sem2, sem3)
    ssems = (ssem0, ssem1, ssem2, ssem3)
    NB = 4
    for b in range(NB):
        pltpu.async_copy(xh.at[row_v.at[b]], bufs[b], gsems[b])

    HALF_K = NCHUNK // NB // 2

    def body(k, carry):
        base = NB * k
        mine = (c == 0) == (k < HALF_K)
        for b in range(NB):
            i = base + b
            pltpu.make_async_copy(xh.at[row_v.at[i]], bufs[b], gsems[b]).wait()
            pltpu.async_copy(bufs[b], agg_sh.at[col_v.at[i]], ssems[b],
                             add=True)

            @pl.when(mine)
            def _():
                pltpu.sync_copy(ones_c, deg_sh.at[col_v.at[i]], add=True)

        @pl.when(k < NCHUNK // NB - 1)
        def _():
            for b in range(NB):
                i = base + b
                pltpu.make_async_copy(
                    bufs[b], agg_sh.at[col_v.at[i]], ssems[b]).wait()
                pltpu.async_copy(xh.at[row_v.at[i + NB]], bufs[b], gsems[b])

        return carry

    lax.fori_loop(0, NCHUNK // NB, body, 0)

    for b in range(NB):
        pltpu.make_async_copy(
            bufs[b], agg_sh.at[col_v.at[NCHUNK - NB + b]], ssems[b]).wait()

    offs = tuple(j * 16 for j in range(CHUNK // 16)) + (CHUNK - 16,)

    def ebody(i, carry):
        e_row = e_base + i * CHUNK
        for o in offs:
            col16 = col_v[i, pl.ds(o, 16)]
            e16 = (e_row + o) + lane
            plsc.store_scatter(elast_v, [col16], e16)
        return carry

    lax.fori_loop(c * (NCHUNK // 2), (c + 1) * (NCHUNK // 2), ebody, 0)
    pltpu.sync_copy(elast_v, elast_out.at[c, s])

    plsc.subcore_barrier()
    pltpu.sync_copy(agg_sh.at[pl.ds(s * ROWS_PER_SUB, ROWS_PER_SUB)],
                    agg_out.at[c, pl.ds(s * ROWS_PER_SUB, ROWS_PER_SUB)])

    @pl.when(s == 0)
    def _():
        pltpu.sync_copy(deg_sh, deg_out.at[c])


_edge_pass = functools.partial(
    pl.kernel,
    mesh=plsc.VectorSubcoreMesh(core_axis_name="c", subcore_axis_name="s"),
    compiler_params=pltpu.CompilerParams(
        needs_layout_passes=False, use_tc_tiling_on_sc=False),
    out_type=[
        jax.ShapeDtypeStruct((NC, N_ACC, D_HALF), jnp.float32),
        jax.ShapeDtypeStruct((NC, N_ACC), jnp.float32),
        jax.ShapeDtypeStruct((NC, NS, N_PAD), jnp.int32),
    ],
    scratch_types=[
        pltpu.VMEM((NCHUNK, CHUNK), jnp.int32),
        pltpu.VMEM((NCHUNK, CHUNK), jnp.int32),
        pltpu.VMEM((CHUNK, D_HALF), jnp.float32),
        pltpu.VMEM((CHUNK, D_HALF), jnp.float32),
        pltpu.VMEM((CHUNK, D_HALF), jnp.float32),
        pltpu.VMEM((CHUNK, D_HALF), jnp.float32),
        pltpu.VMEM((128,), jnp.float32),
        pltpu.VMEM((N_PAD,), jnp.int32),
        pltpu.SemaphoreType.DMA,
        pltpu.SemaphoreType.DMA,
        pltpu.SemaphoreType.DMA,
        pltpu.SemaphoreType.DMA,
        pltpu.SemaphoreType.DMA,
        pltpu.SemaphoreType.DMA,
        pltpu.SemaphoreType.DMA,
        pltpu.SemaphoreType.DMA,
        pltpu.VMEM_SHARED((N_ACC, D_HALF), jnp.float32),
        pltpu.VMEM_SHARED((N_ACC,), jnp.float32),
    ],
)(_edge_pass_body)


def _select_pass_body(elast_hbm, attr_hbm,
                      mask_out, attr_out,
                      tbl_v, idx_v, mask_v, arows_v, sem):
    c = lax.axis_index("c")
    s = lax.axis_index("s")
    wid = c * NS + s
    base = wid * SEL_PER_TEC

    pltpu.sync_copy(elast_hbm.at[:, pl.ds(base, SEL_PER_TEC)], tbl_v)

    def combine(k, carry):
        m = tbl_v[0, pl.ds(k * 16, 16)]
        for j in range(1, NW):
            m = jnp.maximum(m, tbl_v[j, pl.ds(k * 16, 16)])
        idx_v[pl.ds(k * 16, 16)] = jnp.maximum(m, 0)
        mask_v[pl.ds(k * 16, 16)] = jnp.where(m >= 0, 1.0, 0.0)
        return carry

    lax.fori_loop(0, SEL_VREGS, combine, 0)

    pltpu.sync_copy(mask_v, mask_out.at[pl.ds(base, SEL_PER_TEC)])

    def gather(t, carry):
        pltpu.async_copy(
            attr_hbm.at[idx_v.at[pl.ds(t * SEL_CHUNK, SEL_CHUNK)]],
            arows_v.at[pl.ds(t * SEL_CHUNK, SEL_CHUNK)], sem).wait()
        return carry

    lax.fori_loop(0, SEL_PER_TEC // SEL_CHUNK, gather, 0)
    pltpu.sync_copy(arows_v, attr_out.at[pl.ds(base, SEL_PER_TEC)])


_select_pass = functools.partial(
    pl.kernel,
    mesh=plsc.VectorSubcoreMesh(core_axis_name="c", subcore_axis_name="s"),
    compiler_params=pltpu.CompilerParams(
        needs_layout_passes=False, use_tc_tiling_on_sc=False),
    out_type=[
        jax.ShapeDtypeStruct((N_PAD,), jnp.float32),
        jax.ShapeDtypeStruct((N_PAD, D_EDGE), jnp.float32),
    ],
    scratch_types=[
        pltpu.VMEM((NW, SEL_PER_TEC), jnp.int32),
        pltpu.VMEM((SEL_PER_TEC,), jnp.int32),
        pltpu.VMEM((SEL_PER_TEC,), jnp.float32),
        pltpu.VMEM((SEL_PER_TEC, D_EDGE), jnp.float32),
        pltpu.SemaphoreType.DMA,
    ],
)(_select_pass_body)


def _mm(a, b):
    return jax.lax.dot_general(
        a, b, (((1,), (0,)), ((), ())),
        precision=jax.lax.Precision.DEFAULT,
        preferred_element_type=jnp.float32)


BM = 2000
GRID = N // BM


def _dense_a_body(x_ref, agg0_ref, agg1_ref, deg0_ref, deg1_ref,
                  attr_ref, mask_ref, WlT_ref, bl_ref, WrT_ref,
                  WembT_ref, bemb_ref, WteT_ref, bte_ref, WtaT_ref, bta_ref,
                  WgT_ref, bg_ref, o_ref, sum_ref, sq_ref):
    x = x_ref[...]
    deg = jnp.maximum(deg0_ref[...] + deg1_ref[...], 1.0)
    h0 = agg0_ref[...] / deg
    h1 = agg1_ref[...] / deg
    WlT = WlT_ref[...]
    out = (_mm(h0, WlT[:D_HALF]) + _mm(h1, WlT[D_HALF:]) + bl_ref[...]
           + _mm(x, WrT_ref[...]))

    attr = attr_ref[...]
    emb = jnp.maximum(_mm(attr, WembT_ref[...]) + bemb_ref[...], 0.0)
    t_emb = _mm(emb, WteT_ref[...]) + bte_ref[...]
    t_attr = _mm(attr, WtaT_ref[...]) + bta_ref[...]
    t = t_emb + t_attr

    WgT = WgT_ref[...]
    logits = (_mm(out, WgT[:D_OUT]) + _mm(t, WgT[D_OUT:2 * D_OUT])
              + _mm(t_attr, WgT[2 * D_OUT:]) + bg_ref[...])
    gate = 1.0 / (1.0 + jnp.exp(-logits))
    contrib = gate * t * mask_ref[...]

    o = out + contrib
    o_ref[...] = o

    @pl.when(pl.program_id(0) == 0)
    def _():
        sum_ref[...] = jnp.zeros_like(sum_ref)
        sq_ref[...] = jnp.zeros_like(sq_ref)

    sum_ref[...] += jnp.sum(o, axis=0, keepdims=True)
    sq_ref[...] += jnp.sum(o * o, axis=0, keepdims=True)


_row_spec = pl.BlockSpec((BM, D_IN), lambda i: (i, 0))
_half_spec = pl.BlockSpec((BM, D_HALF), lambda i: (i, 0))
_col_spec = pl.BlockSpec((BM, 1), lambda i: (i, 0))


def _w_spec(r, c):
    return pl.BlockSpec((r, c), lambda i: (0, 0))


_dense_a = pl.pallas_call(
    _dense_a_body,
    grid=(GRID,),
    in_specs=[
        _row_spec, _half_spec, _half_spec, _col_spec, _col_spec,
        pl.BlockSpec((BM, D_EDGE), lambda i: (i, 0)), _col_spec,
        _w_spec(D_IN, D_OUT), _w_spec(1, D_OUT), _w_spec(D_IN, D_OUT),
        _w_spec(D_EDGE, D_EDGE), _w_spec(1, D_EDGE),
        _w_spec(D_EDGE, D_OUT), _w_spec(1, D_OUT),
        _w_spec(D_EDGE, D_OUT), _w_spec(1, D_OUT),
        _w_spec(3 * D_OUT, D_OUT), _w_spec(1, D_OUT),
    ],
    out_specs=[
        pl.BlockSpec((BM, D_OUT), lambda i: (i, 0)),
        _w_spec(1, D_OUT), _w_spec(1, D_OUT),
    ],
    out_shape=[
        jax.ShapeDtypeStruct((N, D_OUT), jnp.float32),
        jax.ShapeDtypeStruct((1, D_OUT), jnp.float32),
        jax.ShapeDtypeStruct((1, D_OUT), jnp.float32),
    ],
)


def _dense_b_body(o_ref, sum_ref, sq_ref, gamma_ref, beta_ref, out_ref):
    mu = sum_ref[...] * (1.0 / N)
    var = sq_ref[...] * (1.0 / N) - mu * mu
    o = o_ref[...]
    o = (o - mu) / jnp.sqrt(var + 1e-5) * gamma_ref[...] + beta_ref[...]
    out_ref[...] = jnp.maximum(o + o, 0.0)


_dense_b = pl.pallas_call(
    _dense_b_body,
    grid=(GRID,),
    in_specs=[
        pl.BlockSpec((BM, D_OUT), lambda i: (i, 0)),
        _w_spec(1, D_OUT), _w_spec(1, D_OUT),
        _w_spec(1, D_OUT), _w_spec(1, D_OUT),
    ],
    out_specs=pl.BlockSpec((BM, D_OUT), lambda i: (i, 0)),
    out_shape=jax.ShapeDtypeStruct((N, D_OUT), jnp.float32),
)


def kernel(x, edge_index, edge_attr, W_l, b_l, W_r, W_emb, b_emb,
           W_te, b_te, W_ta, b_ta, W_gate, b_gate, gamma, beta):
    ei = edge_index.astype(jnp.int32).reshape(2, NS, NCHUNK, CHUNK)
    x2 = jnp.stack([x[:, :D_HALF], x[:, D_HALF:]])
    zrow = jnp.zeros((N_ACC, D_HALF), jnp.float32)
    zdeg = jnp.zeros((N_ACC,), jnp.float32)
    neg1 = jnp.full((N_PAD,), -1, jnp.int32)

    agg_p, deg_p, elast = _edge_pass(x2, ei, zrow, zdeg, neg1)
    mask, attr_sel = _select_pass(elast.reshape(NW, N_PAD), edge_attr)

    o, osum, osq = _dense_a(
        x, agg_p[0, :N], agg_p[1, :N],
        deg_p[0, :N].reshape(N, 1), deg_p[1, :N].reshape(N, 1),
        attr_sel[:N], mask[:N].reshape(N, 1),
        W_l.T, b_l.reshape(1, -1), W_r.T,
        W_emb.T, b_emb.reshape(1, -1), W_te.T, b_te.reshape(1, -1),
        W_ta.T, b_ta.reshape(1, -1),
        W_gate.T, b_gate.reshape(1, -1))
    return _dense_b(o, osum, osq, gamma.reshape(1, -1), beta.reshape(1, -1))


# speedup vs baseline: 16.1564x; 1.0133x over previous
import functools

import jax
import jax.numpy as jnp
from jax import lax
from jax.experimental import pallas as pl
from jax.experimental.pallas import tpu as pltpu
from jax.experimental.pallas import tpu_sc as plsc

N = 10000
E = 320000
D_IN = 128
D_OUT = 128
D_EDGE = 16
D_HALF = D_IN // 2

NC = 2
NS = 16
E_PER_TEC = E // NS
CHUNK = 125
NCHUNK = E_PER_TEC // CHUNK
EVREGS = E_PER_TEC // 16
N_ACC = 10240
ROWS_PER_SUB = N_ACC // NS
N_PAD = 10240
NW = NC * NS
SEL_PER_TEC = N_PAD // NW
SEL_VREGS = SEL_PER_TEC // 16
SEL_CHUNK = 80


def _edge_pass_body(x2_hbm, ei_hbm, zrow_hbm, zdeg_hbm,
                    neg1_hbm,
                    agg_out, deg_out, elast_out,
                    row_v, col_v, rows_v0, rows_v1, rows_v2, rows_v3,
                    ones_v, elast_v,
                    sem0, sem1, sem2, sem3, ssem0, ssem1, ssem2, ssem3,
                    agg_sh, deg_sh):
    c = lax.axis_index("c")
    s = lax.axis_index("s")

    pltpu.sync_copy(zrow_hbm.at[pl.ds(s * ROWS_PER_SUB, ROWS_PER_SUB)],
                    agg_sh.at[pl.ds(s * ROWS_PER_SUB, ROWS_PER_SUB)])

    @pl.when(s == 0)
    def _():
        pltpu.sync_copy(zdeg_hbm, deg_sh)

    pltpu.sync_copy(ei_hbm.at[0, s], row_v)
    pltpu.sync_copy(ei_hbm.at[1, s], col_v)

    pltpu.sync_copy(neg1_hbm, elast_v)

    for j in range(8):
        ones_v[pl.ds(j * 16, 16)] = jnp.full((16,), 1.0, jnp.float32)

    plsc.subcore_barrier()

    lane = lax.iota(jnp.int32, 16)
    e_base = s * E_PER_TEC
    xh = x2_hbm.at[c]
    ones_c = ones_v.at[pl.ds(0, CHUNK)]

    bufs = (rows_v0, rows_v1, rows_v2, rows_v3)
    gsems = (sem0, sem1, sem2, sem3)
    ssems = (ssem0, ssem1, ssem2, ssem3)
    NB = 4
    for b in range(NB):
        pltpu.async_copy(xh.at[row_v.at[b]], bufs[b], gsems[b])

    HALF_K = NCHUNK // NB // 2
    offs = tuple(j * 16 for j in range(CHUNK // 16)) + (CHUNK - 16,)

    def body(k, carry):
        base = NB * k
        mine = (c == 0) == (k < HALF_K)
        for b in range(NB):
            i = base + b
            pltpu.make_async_copy(xh.at[row_v.at[i]], bufs[b], gsems[b]).wait()
            pltpu.async_copy(bufs[b], agg_sh.at[col_v.at[i]], ssems[b],
                             add=True)

            @pl.when(mine)
            def _():
                pltpu.sync_copy(ones_c, deg_sh.at[col_v.at[i]], add=True)
                e_row = e_base + i * CHUNK
                for o in offs:
                    col16 = col_v[i, pl.ds(o, 16)]
                    plsc.store_scatter(elast_v, [col16], (e_row + o) + lane)

        @pl.when(k < NCHUNK // NB - 1)
        def _():
            for b in range(NB):
                i = base + b
                pltpu.make_async_copy(
                    bufs[b], agg_sh.at[col_v.at[i]], ssems[b]).wait()
                pltpu.async_copy(xh.at[row_v.at[i + NB]], bufs[b], gsems[b])

        return carry

    lax.fori_loop(0, NCHUNK // NB, body, 0)

    for b in range(NB):
        pltpu.make_async_copy(
            bufs[b], agg_sh.at[col_v.at[NCHUNK - NB + b]], ssems[b]).wait()

    pltpu.sync_copy(elast_v, elast_out.at[c, s])

    plsc.subcore_barrier()
    pltpu.sync_copy(agg_sh.at[pl.ds(s * ROWS_PER_SUB, ROWS_PER_SUB)],
                    agg_out.at[c, pl.ds(s * ROWS_PER_SUB, ROWS_PER_SUB)])

    @pl.when(s == 0)
    def _():
        pltpu.sync_copy(deg_sh, deg_out.at[c])


_edge_pass = functools.partial(
    pl.kernel,
    mesh=plsc.VectorSubcoreMesh(core_axis_name="c", subcore_axis_name="s"),
    compiler_params=pltpu.CompilerParams(
        needs_layout_passes=False, use_tc_tiling_on_sc=False),
    out_type=[
        jax.ShapeDtypeStruct((NC, N_ACC, D_HALF), jnp.float32),
        jax.ShapeDtypeStruct((NC, N_ACC), jnp.float32),
        jax.ShapeDtypeStruct((NC, NS, N_PAD), jnp.int32),
    ],
    scratch_types=[
        pltpu.VMEM((NCHUNK, CHUNK), jnp.int32),
        pltpu.VMEM((NCHUNK, CHUNK), jnp.int32),
        pltpu.VMEM((CHUNK, D_HALF), jnp.float32),
        pltpu.VMEM((CHUNK, D_HALF), jnp.float32),
        pltpu.VMEM((CHUNK, D_HALF), jnp.float32),
        pltpu.VMEM((CHUNK, D_HALF), jnp.float32),
        pltpu.VMEM((128,), jnp.float32),
        pltpu.VMEM((N_PAD,), jnp.int32),
        pltpu.SemaphoreType.DMA,
        pltpu.SemaphoreType.DMA,
        pltpu.SemaphoreType.DMA,
        pltpu.SemaphoreType.DMA,
        pltpu.SemaphoreType.DMA,
        pltpu.SemaphoreType.DMA,
        pltpu.SemaphoreType.DMA,
        pltpu.SemaphoreType.DMA,
        pltpu.VMEM_SHARED((N_ACC, D_HALF), jnp.float32),
        pltpu.VMEM_SHARED((N_ACC,), jnp.float32),
    ],
)(_edge_pass_body)


def _select_pass_body(elast_hbm, attr_hbm,
                      mask_out, attr_out,
                      tbl_v, idx_v, mask_v, arows_v, sem):
    c = lax.axis_index("c")
    s = lax.axis_index("s")
    wid = c * NS + s
    base = wid * SEL_PER_TEC

    pltpu.sync_copy(elast_hbm.at[:, pl.ds(base, SEL_PER_TEC)], tbl_v)

    def combine(k, carry):
        m = tbl_v[0, pl.ds(k * 16, 16)]
        for j in range(1, NW):
            m = jnp.maximum(m, tbl_v[j, pl.ds(k * 16, 16)])
        idx_v[pl.ds(k * 16, 16)] = jnp.maximum(m, 0)
        mask_v[pl.ds(k * 16, 16)] = jnp.where(m >= 0, 1.0, 0.0)
        return carry

    lax.fori_loop(0, SEL_VREGS, combine, 0)

    pltpu.sync_copy(mask_v, mask_out.at[pl.ds(base, SEL_PER_TEC)])

    def gather(t, carry):
        pltpu.async_copy(
            attr_hbm.at[idx_v.at[pl.ds(t * SEL_CHUNK, SEL_CHUNK)]],
            arows_v.at[pl.ds(t * SEL_CHUNK, SEL_CHUNK)], sem).wait()
        return carry

    lax.fori_loop(0, SEL_PER_TEC // SEL_CHUNK, gather, 0)
    pltpu.sync_copy(arows_v, attr_out.at[pl.ds(base, SEL_PER_TEC)])


_select_pass = functools.partial(
    pl.kernel,
    mesh=plsc.VectorSubcoreMesh(core_axis_name="c", subcore_axis_name="s"),
    compiler_params=pltpu.CompilerParams(
        needs_layout_passes=False, use_tc_tiling_on_sc=False),
    out_type=[
        jax.ShapeDtypeStruct((N_PAD,), jnp.float32),
        jax.ShapeDtypeStruct((N_PAD, D_EDGE), jnp.float32),
    ],
    scratch_types=[
        pltpu.VMEM((NW, SEL_PER_TEC), jnp.int32),
        pltpu.VMEM((SEL_PER_TEC,), jnp.int32),
        pltpu.VMEM((SEL_PER_TEC,), jnp.float32),
        pltpu.VMEM((SEL_PER_TEC, D_EDGE), jnp.float32),
        pltpu.SemaphoreType.DMA,
    ],
)(_select_pass_body)


def _mm(a, b):
    return jax.lax.dot_general(
        a, b, (((1,), (0,)), ((), ())),
        precision=jax.lax.Precision.DEFAULT,
        preferred_element_type=jnp.float32)


BM = 2000
GRID = N // BM


def _dense_a_body(x_ref, agg0_ref, agg1_ref, deg0_ref, deg1_ref,
                  attr_ref, mask_ref, WlT_ref, bl_ref, WrT_ref,
                  WembT_ref, bemb_ref, WteT_ref, bte_ref, WtaT_ref, bta_ref,
                  WgT_ref, bg_ref, o_ref, sum_ref, sq_ref):
    x = x_ref[...]
    deg = jnp.maximum(deg0_ref[...] + deg1_ref[...], 1.0)
    h0 = agg0_ref[...] / deg
    h1 = agg1_ref[...] / deg
    WlT = WlT_ref[...]
    out = (_mm(h0, WlT[:D_HALF]) + _mm(h1, WlT[D_HALF:]) + bl_ref[...]
           + _mm(x, WrT_ref[...]))

    attr = attr_ref[...]
    emb = jnp.maximum(_mm(attr, WembT_ref[...]) + bemb_ref[...], 0.0)
    t_emb = _mm(emb, WteT_ref[...]) + bte_ref[...]
    t_attr = _mm(attr, WtaT_ref[...]) + bta_ref[...]
    t = t_emb + t_attr

    WgT = WgT_ref[...]
    logits = (_mm(out, WgT[:D_OUT]) + _mm(t, WgT[D_OUT:2 * D_OUT])
              + _mm(t_attr, WgT[2 * D_OUT:]) + bg_ref[...])
    gate = 1.0 / (1.0 + jnp.exp(-logits))
    contrib = gate * t * mask_ref[...]

    o = out + contrib
    o_ref[...] = o

    @pl.when(pl.program_id(0) == 0)
    def _():
        sum_ref[...] = jnp.zeros_like(sum_ref)
        sq_ref[...] = jnp.zeros_like(sq_ref)

    sum_ref[...] += jnp.sum(o, axis=0, keepdims=True)
    sq_ref[...] += jnp.sum(o * o, axis=0, keepdims=True)


_row_spec = pl.BlockSpec((BM, D_IN), lambda i: (i, 0))
_half_spec = pl.BlockSpec((BM, D_HALF), lambda i: (i, 0))
_col_spec = pl.BlockSpec((BM, 1), lambda i: (i, 0))


def _w_spec(r, c):
    return pl.BlockSpec((r, c), lambda i: (0, 0))


_dense_a = pl.pallas_call(
    _dense_a_body,
    grid=(GRID,),
    in_specs=[
        _row_spec, _half_spec, _half_spec, _col_spec, _col_spec,
        pl.BlockSpec((BM, D_EDGE), lambda i: (i, 0)), _col_spec,
        _w_spec(D_IN, D_OUT), _w_spec(1, D_OUT), _w_spec(D_IN, D_OUT),
        _w_spec(D_EDGE, D_EDGE), _w_spec(1, D_EDGE),
        _w_spec(D_EDGE, D_OUT), _w_spec(1, D_OUT),
        _w_spec(D_EDGE, D_OUT), _w_spec(1, D_OUT),
        _w_spec(3 * D_OUT, D_OUT), _w_spec(1, D_OUT),
    ],
    out_specs=[
        pl.BlockSpec((BM, D_OUT), lambda i: (i, 0)),
        _w_spec(1, D_OUT), _w_spec(1, D_OUT),
    ],
    out_shape=[
        jax.ShapeDtypeStruct((N, D_OUT), jnp.float32),
        jax.ShapeDtypeStruct((1, D_OUT), jnp.float32),
        jax.ShapeDtypeStruct((1, D_OUT), jnp.float32),
    ],
)


def _dense_b_body(o_ref, sum_ref, sq_ref, gamma_ref, beta_ref, out_ref):
    mu = sum_ref[...] * (1.0 / N)
    var = sq_ref[...] * (1.0 / N) - mu * mu
    o = o_ref[...]
    o = (o - mu) / jnp.sqrt(var + 1e-5) * gamma_ref[...] + beta_ref[...]
    out_ref[...] = jnp.maximum(o + o, 0.0)


_dense_b = pl.pallas_call(
    _dense_b_body,
    grid=(GRID,),
    in_specs=[
        pl.BlockSpec((BM, D_OUT), lambda i: (i, 0)),
        _w_spec(1, D_OUT), _w_spec(1, D_OUT),
        _w_spec(1, D_OUT), _w_spec(1, D_OUT),
    ],
    out_specs=pl.BlockSpec((BM, D_OUT), lambda i: (i, 0)),
    out_shape=jax.ShapeDtypeStruct((N, D_OUT), jnp.float32),
)


def kernel(x, edge_index, edge_attr, W_l, b_l, W_r, W_emb, b_emb,
           W_te, b_te, W_ta, b_ta, W_gate, b_gate, gamma, beta):
    ei = edge_index.astype(jnp.int32).reshape(2, NS, NCHUNK, CHUNK)
    x2 = jnp.stack([x[:, :D_HALF], x[:, D_HALF:]])
    zrow = jnp.zeros((N_ACC, D_HALF), jnp.float32)
    zdeg = jnp.zeros((N_ACC,), jnp.float32)
    neg1 = jnp.full((N_PAD,), -1, jnp.int32)

    agg_p, deg_p, elast = _edge_pass(x2, ei, zrow, zdeg, neg1)
    mask, attr_sel = _select_pass(elast.reshape(NW, N_PAD), edge_attr)

    o, osum, osq = _dense_a(
        x, agg_p[0, :N], agg_p[1, :N],
        deg_p[0, :N].reshape(N, 1), deg_p[1, :N].reshape(N, 1),
        attr_sel[:N], mask[:N].reshape(N, 1),
        W_l.T, b_l.reshape(1, -1), W_r.T,
        W_emb.T, b_emb.reshape(1, -1), W_te.T, b_te.reshape(1, -1),
        W_ta.T, b_ta.reshape(1, -1),
        W_gate.T, b_gate.reshape(1, -1))
    return _dense_b(o, osum, osq, gamma.reshape(1, -1), beta.reshape(1, -1))


# speedup vs baseline: 16.9236x; 1.0475x over previous
import functools

import jax
import jax.numpy as jnp
from jax import lax
from jax.experimental import pallas as pl
from jax.experimental.pallas import tpu as pltpu
from jax.experimental.pallas import tpu_sc as plsc

N = 10000
E = 320000
D_IN = 128
D_OUT = 128
D_EDGE = 16
D_HALF = D_IN // 2

NC = 2
NS = 16
E_PER_TEC = E // NS
CHUNK = 125
NCHUNK = E_PER_TEC // CHUNK
EVREGS = E_PER_TEC // 16
N_ACC = 10240
ROWS_PER_SUB = N_ACC // NS
N_PAD = 10240
NW = NC * NS
SEL_PER_TEC = N_PAD // NW
SEL_VREGS = SEL_PER_TEC // 16
SEL_CHUNK = 80


def _edge_pass_body(x2_hbm, ei_hbm, zrow_hbm, zdeg_hbm,
                    neg1_hbm,
                    agg_out, deg_out, elast_out,
                    row_v, col_v, rows_v0, rows_v1, rows_v2, rows_v3,
                    ones_v, elast_v,
                    sem0, sem1, sem2, sem3, ssem0, ssem1, ssem2, ssem3, dsem,
                    agg_sh, deg_sh):
    c = lax.axis_index("c")
    s = lax.axis_index("s")

    pltpu.sync_copy(zrow_hbm.at[pl.ds(s * ROWS_PER_SUB, ROWS_PER_SUB)],
                    agg_sh.at[pl.ds(s * ROWS_PER_SUB, ROWS_PER_SUB)])

    @pl.when(s == 0)
    def _():
        pltpu.sync_copy(zdeg_hbm, deg_sh)

    pltpu.sync_copy(ei_hbm.at[0, s], row_v)
    pltpu.sync_copy(ei_hbm.at[1, s], col_v)

    pltpu.sync_copy(neg1_hbm, elast_v)

    for j in range(8):
        ones_v[pl.ds(j * 16, 16)] = jnp.full((16,), 1.0, jnp.float32)

    plsc.subcore_barrier()

    lane = lax.iota(jnp.int32, 16)
    e_base = s * E_PER_TEC
    xh = x2_hbm.at[c]
    ones_c = ones_v.at[pl.ds(0, CHUNK)]

    bufs = (rows_v0, rows_v1, rows_v2, rows_v3)
    gsems = (sem0, sem1, sem2, sem3)
    ssems = (ssem0, ssem1, ssem2, ssem3)
    NB = 4
    for b in range(NB):
        pltpu.async_copy(xh.at[row_v.at[b]], bufs[b], gsems[b])

    HALF_K = NCHUNK // NB // 2
    offs = tuple(j * 16 for j in range(CHUNK // 16)) + (CHUNK - 16,)

    def body(k, carry):
        base = NB * k
        mine = (c == 0) == (k < HALF_K)
        for b in range(NB):
            i = base + b
            pltpu.make_async_copy(xh.at[row_v.at[i]], bufs[b], gsems[b]).wait()
            pltpu.async_copy(bufs[b], agg_sh.at[col_v.at[i]], ssems[b],
                             add=True)

            @pl.when(mine)
            def _():
                pltpu.async_copy(ones_c, deg_sh.at[col_v.at[i]], dsem,
                                 add=True)
                e_row = e_base + i * CHUNK
                for o in offs:
                    col16 = col_v[i, pl.ds(o, 16)]
                    plsc.store_scatter(elast_v, [col16], (e_row + o) + lane)

        @pl.when(k < NCHUNK // NB - 1)
        def _():
            for b in range(NB):
                i = base + b
                pltpu.make_async_copy(
                    bufs[b], agg_sh.at[col_v.at[i]], ssems[b]).wait()
                pltpu.async_copy(xh.at[row_v.at[i + NB]], bufs[b], gsems[b])

        return carry

    lax.fori_loop(0, NCHUNK // NB, body, 0)

    for b in range(NB):
        pltpu.make_async_copy(
            bufs[b], agg_sh.at[col_v.at[NCHUNK - NB + b]], ssems[b]).wait()

    def ddrain(i, carry):
        pltpu.make_async_copy(ones_c, deg_sh.at[col_v.at[i]], dsem).wait()
        return carry

    lax.fori_loop(c * (NCHUNK // 2), (c + 1) * (NCHUNK // 2), ddrain, 0)

    pltpu.sync_copy(elast_v, elast_out.at[c, s])

    plsc.subcore_barrier()
    pltpu.sync_copy(agg_sh.at[pl.ds(s * ROWS_PER_SUB, ROWS_PER_SUB)],
                    agg_out.at[c, pl.ds(s * ROWS_PER_SUB, ROWS_PER_SUB)])

    @pl.when(s == 0)
    def _():
        pltpu.sync_copy(deg_sh, deg_out.at[c])


_edge_pass = functools.partial(
    pl.kernel,
    mesh=plsc.VectorSubcoreMesh(core_axis_name="c", subcore_axis_name="s"),
    compiler_params=pltpu.CompilerParams(
        needs_layout_passes=False, use_tc_tiling_on_sc=False),
    out_type=[
        jax.ShapeDtypeStruct((NC, N_ACC, D_HALF), jnp.float32),
        jax.ShapeDtypeStruct((NC, N_ACC), jnp.float32),
        jax.ShapeDtypeStruct((NC, NS, N_PAD), jnp.int32),
    ],
    scratch_types=[
        pltpu.VMEM((NCHUNK, CHUNK), jnp.int32),
        pltpu.VMEM((NCHUNK, CHUNK), jnp.int32),
        pltpu.VMEM((CHUNK, D_HALF), jnp.float32),
        pltpu.VMEM((CHUNK, D_HALF), jnp.float32),
        pltpu.VMEM((CHUNK, D_HALF), jnp.float32),
        pltpu.VMEM((CHUNK, D_HALF), jnp.float32),
        pltpu.VMEM((128,), jnp.float32),
        pltpu.VMEM((N_PAD,), jnp.int32),
        pltpu.SemaphoreType.DMA,
        pltpu.SemaphoreType.DMA,
        pltpu.SemaphoreType.DMA,
        pltpu.SemaphoreType.DMA,
        pltpu.SemaphoreType.DMA,
        pltpu.SemaphoreType.DMA,
        pltpu.SemaphoreType.DMA,
        pltpu.SemaphoreType.DMA,
        pltpu.SemaphoreType.DMA,
        pltpu.VMEM_SHARED((N_ACC, D_HALF), jnp.float32),
        pltpu.VMEM_SHARED((N_ACC,), jnp.float32),
    ],
)(_edge_pass_body)


def _select_pass_body(elast_hbm, attr_hbm,
                      mask_out, attr_out,
                      tbl_v, idx_v, mask_v, arows_v, sem):
    c = lax.axis_index("c")
    s = lax.axis_index("s")
    wid = c * NS + s
    base = wid * SEL_PER_TEC

    pltpu.sync_copy(elast_hbm.at[:, pl.ds(base, SEL_PER_TEC)], tbl_v)

    def combine(k, carry):
        m = tbl_v[0, pl.ds(k * 16, 16)]
        for j in range(1, NW):
            m = jnp.maximum(m, tbl_v[j, pl.ds(k * 16, 16)])
        idx_v[pl.ds(k * 16, 16)] = jnp.maximum(m, 0)
        mask_v[pl.ds(k * 16, 16)] = jnp.where(m >= 0, 1.0, 0.0)
        return carry

    lax.fori_loop(0, SEL_VREGS, combine, 0)

    pltpu.sync_copy(mask_v, mask_out.at[pl.ds(base, SEL_PER_TEC)])

    def gather(t, carry):
        pltpu.async_copy(
            attr_hbm.at[idx_v.at[pl.ds(t * SEL_CHUNK, SEL_CHUNK)]],
            arows_v.at[pl.ds(t * SEL_CHUNK, SEL_CHUNK)], sem).wait()
        return carry

    lax.fori_loop(0, SEL_PER_TEC // SEL_CHUNK, gather, 0)
    pltpu.sync_copy(arows_v, attr_out.at[pl.ds(base, SEL_PER_TEC)])


_select_pass = functools.partial(
    pl.kernel,
    mesh=plsc.VectorSubcoreMesh(core_axis_name="c", subcore_axis_name="s"),
    compiler_params=pltpu.CompilerParams(
        needs_layout_passes=False, use_tc_tiling_on_sc=False),
    out_type=[
        jax.ShapeDtypeStruct((N_PAD,), jnp.float32),
        jax.ShapeDtypeStruct((N_PAD, D_EDGE), jnp.float32),
    ],
    scratch_types=[
        pltpu.VMEM((NW, SEL_PER_TEC), jnp.int32),
        pltpu.VMEM((SEL_PER_TEC,), jnp.int32),
        pltpu.VMEM((SEL_PER_TEC,), jnp.float32),
        pltpu.VMEM((SEL_PER_TEC, D_EDGE), jnp.float32),
        pltpu.SemaphoreType.DMA,
    ],
)(_select_pass_body)


def _mm(a, b):
    return jax.lax.dot_general(
        a, b, (((1,), (0,)), ((), ())),
        precision=jax.lax.Precision.DEFAULT,
        preferred_element_type=jnp.float32)


BM = 2000
GRID = N // BM


def _dense_a_body(x_ref, agg0_ref, agg1_ref, deg0_ref, deg1_ref,
                  attr_ref, mask_ref, WlT_ref, bl_ref, WrT_ref,
                  WembT_ref, bemb_ref, WteT_ref, bte_ref, WtaT_ref, bta_ref,
                  WgT_ref, bg_ref, o_ref, sum_ref, sq_ref):
    x = x_ref[...]
    deg = jnp.maximum(deg0_ref[...] + deg1_ref[...], 1.0)
    h0 = agg0_ref[...] / deg
    h1 = agg1_ref[...] / deg
    WlT = WlT_ref[...]
    out = (_mm(h0, WlT[:D_HALF]) + _mm(h1, WlT[D_HALF:]) + bl_ref[...]
           + _mm(x, WrT_ref[...]))

    attr = attr_ref[...]
    emb = jnp.maximum(_mm(attr, WembT_ref[...]) + bemb_ref[...], 0.0)
    t_emb = _mm(emb, WteT_ref[...]) + bte_ref[...]
    t_attr = _mm(attr, WtaT_ref[...]) + bta_ref[...]
    t = t_emb + t_attr

    WgT = WgT_ref[...]
    logits = (_mm(out, WgT[:D_OUT]) + _mm(t, WgT[D_OUT:2 * D_OUT])
              + _mm(t_attr, WgT[2 * D_OUT:]) + bg_ref[...])
    gate = 1.0 / (1.0 + jnp.exp(-logits))
    contrib = gate * t * mask_ref[...]

    o = out + contrib
    o_ref[...] = o

    @pl.when(pl.program_id(0) == 0)
    def _():
        sum_ref[...] = jnp.zeros_like(sum_ref)
        sq_ref[...] = jnp.zeros_like(sq_ref)

    sum_ref[...] += jnp.sum(o, axis=0, keepdims=True)
    sq_ref[...] += jnp.sum(o * o, axis=0, keepdims=True)


_row_spec = pl.BlockSpec((BM, D_IN), lambda i: (i, 0))
_half_spec = pl.BlockSpec((BM, D_HALF), lambda i: (i, 0))
_col_spec = pl.BlockSpec((BM, 1), lambda i: (i, 0))


def _w_spec(r, c):
    return pl.BlockSpec((r, c), lambda i: (0, 0))


_dense_a = pl.pallas_call(
    _dense_a_body,
    grid=(GRID,),
    in_specs=[
        _row_spec, _half_spec, _half_spec, _col_spec, _col_spec,
        pl.BlockSpec((BM, D_EDGE), lambda i: (i, 0)), _col_spec,
        _w_spec(D_IN, D_OUT), _w_spec(1, D_OUT), _w_spec(D_IN, D_OUT),
        _w_spec(D_EDGE, D_EDGE), _w_spec(1, D_EDGE),
        _w_spec(D_EDGE, D_OUT), _w_spec(1, D_OUT),
        _w_spec(D_EDGE, D_OUT), _w_spec(1, D_OUT),
        _w_spec(3 * D_OUT, D_OUT), _w_spec(1, D_OUT),
    ],
    out_specs=[
        pl.BlockSpec((BM, D_OUT), lambda i: (i, 0)),
        _w_spec(1, D_OUT), _w_spec(1, D_OUT),
    ],
    out_shape=[
        jax.ShapeDtypeStruct((N, D_OUT), jnp.float32),
        jax.ShapeDtypeStruct((1, D_OUT), jnp.float32),
        jax.ShapeDtypeStruct((1, D_OUT), jnp.float32),
    ],
)


def _dense_b_body(o_ref, sum_ref, sq_ref, gamma_ref, beta_ref, out_ref):
    mu = sum_ref[...] * (1.0 / N)
    var = sq_ref[...] * (1.0 / N) - mu * mu
    o = o_ref[...]
    o = (o - mu) / jnp.sqrt(var + 1e-5) * gamma_ref[...] + beta_ref[...]
    out_ref[...] = jnp.maximum(o + o, 0.0)


_dense_b = pl.pallas_call(
    _dense_b_body,
    grid=(GRID,),
    in_specs=[
        pl.BlockSpec((BM, D_OUT), lambda i: (i, 0)),
        _w_spec(1, D_OUT), _w_spec(1, D_OUT),
        _w_spec(1, D_OUT), _w_spec(1, D_OUT),
    ],
    out_specs=pl.BlockSpec((BM, D_OUT), lambda i: (i, 0)),
    out_shape=jax.ShapeDtypeStruct((N, D_OUT), jnp.float32),
)


def kernel(x, edge_index, edge_attr, W_l, b_l, W_r, W_emb, b_emb,
           W_te, b_te, W_ta, b_ta, W_gate, b_gate, gamma, beta):
    ei = edge_index.astype(jnp.int32).reshape(2, NS, NCHUNK, CHUNK)
    x2 = jnp.stack([x[:, :D_HALF], x[:, D_HALF:]])
    zrow = jnp.zeros((N_ACC, D_HALF), jnp.float32)
    zdeg = jnp.zeros((N_ACC,), jnp.float32)
    neg1 = jnp.full((N_PAD,), -1, jnp.int32)

    agg_p, deg_p, elast = _edge_pass(x2, ei, zrow, zdeg, neg1)
    mask, attr_sel = _select_pass(elast.reshape(NW, N_PAD), edge_attr)

    o, osum, osq = _dense_a(
        x, agg_p[0, :N], agg_p[1, :N],
        deg_p[0, :N].reshape(N, 1), deg_p[1, :N].reshape(N, 1),
        attr_sel[:N], mask[:N].reshape(N, 1),
        W_l.T, b_l.reshape(1, -1), W_r.T,
        W_emb.T, b_emb.reshape(1, -1), W_te.T, b_te.reshape(1, -1),
        W_ta.T, b_ta.reshape(1, -1),
        W_gate.T, b_gate.reshape(1, -1))
    return _dense_b(o, osum, osq, gamma.reshape(1, -1), beta.reshape(1, -1))


# speedup vs baseline: 17.2320x; 1.0182x over previous
import functools

import jax
import jax.numpy as jnp
from jax import lax
from jax.experimental import pallas as pl
from jax.experimental.pallas import tpu as pltpu
from jax.experimental.pallas import tpu_sc as plsc

N = 10000
E = 320000
D_IN = 128
D_OUT = 128
D_EDGE = 16
D_HALF = D_IN // 2

NC = 2
NS = 16
E_PER_TEC = E // NS
CHUNK = 125
NCHUNK = E_PER_TEC // CHUNK
EVREGS = E_PER_TEC // 16
N_ACC = 10240
ROWS_PER_SUB = N_ACC // NS
N_PAD = 10240
NW = NC * NS
SEL_PER_TEC = N_PAD // NW
SEL_VREGS = SEL_PER_TEC // 16
SEL_CHUNK = 80


def _edge_pass_body(x2_hbm, ei_hbm, zrow_hbm, zdeg_hbm,
                    neg1_hbm,
                    agg_out, deg_out, elast_out,
                    row_v, col_v, rows_v0, rows_v1, rows_v2, rows_v3,
                    ones_v, elast_v,
                    sem0, sem1, sem2, sem3, ssem0, ssem1, ssem2, ssem3, dsem,
                    agg_sh, deg_sh):
    c = lax.axis_index("c")
    s = lax.axis_index("s")

    pltpu.sync_copy(zrow_hbm.at[pl.ds(s * ROWS_PER_SUB, ROWS_PER_SUB)],
                    agg_sh.at[pl.ds(s * ROWS_PER_SUB, ROWS_PER_SUB)])

    @pl.when(s == 0)
    def _():
        pltpu.sync_copy(zdeg_hbm, deg_sh)

    pltpu.sync_copy(ei_hbm.at[0, s], row_v)
    pltpu.sync_copy(ei_hbm.at[1, s], col_v)

    pltpu.sync_copy(neg1_hbm, elast_v)

    for j in range(8):
        ones_v[pl.ds(j * 16, 16)] = jnp.full((16,), 1.0, jnp.float32)

    plsc.subcore_barrier()

    lane = lax.iota(jnp.int32, 16)
    e_base = s * E_PER_TEC
    xh = x2_hbm.at[c]
    ones_c = ones_v.at[pl.ds(0, CHUNK)]

    bufs = (rows_v0, rows_v1, rows_v2, rows_v3)
    gsems = (sem0, sem1, sem2, sem3)
    ssems = (ssem0, ssem1, ssem2, ssem3)
    NB = 4
    for b in range(NB):
        pltpu.async_copy(xh.at[row_v.at[b]], bufs[b], gsems[b])

    HALF_K = NCHUNK // NB // 2
    offs = tuple(j * 16 for j in range(CHUNK // 16)) + (CHUNK - 16,)

    def body(k, carry):
        base = NB * k
        mine = (c == 0) == (k < HALF_K)
        for b in range(NB):
            i = base + b
            pltpu.make_async_copy(xh.at[row_v.at[i]], bufs[b], gsems[b]).wait()
            pltpu.async_copy(bufs[b], agg_sh.at[col_v.at[i]], ssems[b],
                             add=True)

            @pl.when(mine)
            def _():
                pltpu.async_copy(ones_c, deg_sh.at[col_v.at[i]], dsem,
                                 add=True)
                e_row = e_base + i * CHUNK
                for o in offs:
                    col16 = col_v[i, pl.ds(o, 16)]
                    plsc.store_scatter(elast_v, [col16], (e_row + o) + lane)

        @pl.when(k < NCHUNK // NB - 1)
        def _():
            for b in range(NB):
                i = base + b
                pltpu.make_async_copy(
                    bufs[b], agg_sh.at[col_v.at[i]], ssems[b]).wait()
                pltpu.async_copy(xh.at[row_v.at[i + NB]], bufs[b], gsems[b])

        return carry

    lax.fori_loop(0, NCHUNK // NB, body, 0)

    for b in range(NB):
        pltpu.make_async_copy(
            bufs[b], agg_sh.at[col_v.at[NCHUNK - NB + b]], ssems[b]).wait()

    def ddrain(i, carry):
        pltpu.make_async_copy(ones_c, deg_sh.at[col_v.at[i]], dsem).wait()
        return carry

    lax.fori_loop(c * (NCHUNK // 2), (c + 1) * (NCHUNK // 2), ddrain, 0)

    pltpu.sync_copy(elast_v, elast_out.at[c, s])

    plsc.subcore_barrier()
    pltpu.sync_copy(agg_sh.at[pl.ds(s * ROWS_PER_SUB, ROWS_PER_SUB)],
                    agg_out.at[c, pl.ds(s * ROWS_PER_SUB, ROWS_PER_SUB)])

    @pl.when(s == 0)
    def _():
        pltpu.sync_copy(deg_sh, deg_out.at[c])


_edge_pass = functools.partial(
    pl.kernel,
    mesh=plsc.VectorSubcoreMesh(core_axis_name="c", subcore_axis_name="s"),
    compiler_params=pltpu.CompilerParams(
        needs_layout_passes=False, use_tc_tiling_on_sc=False),
    out_type=[
        jax.ShapeDtypeStruct((NC, N_ACC, D_HALF), jnp.float32),
        jax.ShapeDtypeStruct((NC, N_ACC), jnp.float32),
        jax.ShapeDtypeStruct((NC, NS, N_PAD), jnp.int32),
    ],
    scratch_types=[
        pltpu.VMEM((NCHUNK, CHUNK), jnp.int32),
        pltpu.VMEM((NCHUNK, CHUNK), jnp.int32),
        pltpu.VMEM((CHUNK, D_HALF), jnp.float32),
        pltpu.VMEM((CHUNK, D_HALF), jnp.float32),
        pltpu.VMEM((CHUNK, D_HALF), jnp.float32),
        pltpu.VMEM((CHUNK, D_HALF), jnp.float32),
        pltpu.VMEM((128,), jnp.float32),
        pltpu.VMEM((N_PAD,), jnp.int32),
        pltpu.SemaphoreType.DMA,
        pltpu.SemaphoreType.DMA,
        pltpu.SemaphoreType.DMA,
        pltpu.SemaphoreType.DMA,
        pltpu.SemaphoreType.DMA,
        pltpu.SemaphoreType.DMA,
        pltpu.SemaphoreType.DMA,
        pltpu.SemaphoreType.DMA,
        pltpu.SemaphoreType.DMA,
        pltpu.VMEM_SHARED((N_ACC, D_HALF), jnp.float32),
        pltpu.VMEM_SHARED((N_ACC,), jnp.float32),
    ],
)(_edge_pass_body)


def _select_pass_body(elast_hbm, attr_hbm,
                      mask_out, attr_out,
                      tbl_v, idx_v, mask_v, arows_v, sem):
    c = lax.axis_index("c")
    s = lax.axis_index("s")
    wid = c * NS + s
    base = wid * SEL_PER_TEC

    pltpu.sync_copy(elast_hbm.at[:, pl.ds(base, SEL_PER_TEC)], tbl_v)

    def combine(k, carry):
        m = tbl_v[0, pl.ds(k * 16, 16)]
        for j in range(1, NW):
            m = jnp.maximum(m, tbl_v[j, pl.ds(k * 16, 16)])
        idx_v[pl.ds(k * 16, 16)] = jnp.maximum(m, 0)
        mask_v[pl.ds(k * 16, 16)] = jnp.where(m >= 0, 1.0, 0.0)
        return carry

    lax.fori_loop(0, SEL_VREGS, combine, 0)

    pltpu.sync_copy(mask_v, mask_out.at[pl.ds(base, SEL_PER_TEC)])

    def gather(t, carry):
        pltpu.async_copy(
            attr_hbm.at[idx_v.at[pl.ds(t * SEL_CHUNK, SEL_CHUNK)]],
            arows_v.at[pl.ds(t * SEL_CHUNK, SEL_CHUNK)], sem).wait()
        return carry

    lax.fori_loop(0, SEL_PER_TEC // SEL_CHUNK, gather, 0)
    pltpu.sync_copy(arows_v, attr_out.at[pl.ds(base, SEL_PER_TEC)])


_select_pass = functools.partial(
    pl.kernel,
    mesh=plsc.VectorSubcoreMesh(core_axis_name="c", subcore_axis_name="s"),
    compiler_params=pltpu.CompilerParams(
        needs_layout_passes=False, use_tc_tiling_on_sc=False),
    out_type=[
        jax.ShapeDtypeStruct((N_PAD,), jnp.float32),
        jax.ShapeDtypeStruct((N_PAD, D_EDGE), jnp.float32),
    ],
    scratch_types=[
        pltpu.VMEM((NW, SEL_PER_TEC), jnp.int32),
        pltpu.VMEM((SEL_PER_TEC,), jnp.int32),
        pltpu.VMEM((SEL_PER_TEC,), jnp.float32),
        pltpu.VMEM((SEL_PER_TEC, D_EDGE), jnp.float32),
        pltpu.SemaphoreType.DMA,
    ],
)(_select_pass_body)


def _mm(a, b):
    return jax.lax.dot_general(
        a, b, (((1,), (0,)), ((), ())),
        precision=jax.lax.Precision.DEFAULT,
        preferred_element_type=jnp.float32)


BM = 2000
GRID = N // BM


def _dense_a_body(x_ref, agg0_ref, agg1_ref, deg0_ref, deg1_ref,
                  attr_ref, mask_ref, WlT_ref, bl_ref, WrT_ref,
                  WembT_ref, bemb_ref, WteT_ref, bte_ref, WtaT_ref, bta_ref,
                  WgT_ref, bg_ref, o_ref, sum_ref, sq_ref):
    x = x_ref[...]
    deg = jnp.maximum(deg0_ref[...] + deg1_ref[...], 1.0)
    h0 = agg0_ref[...] / deg
    h1 = agg1_ref[...] / deg
    WlT = WlT_ref[...]
    out = (_mm(h0, WlT[:D_HALF]) + _mm(h1, WlT[D_HALF:]) + bl_ref[...]
           + _mm(x, WrT_ref[...]))

    attr = attr_ref[...]
    emb = jnp.maximum(_mm(attr, WembT_ref[...]) + bemb_ref[...], 0.0)
    t_emb = _mm(emb, WteT_ref[...]) + bte_ref[...]
    t_attr = _mm(attr, WtaT_ref[...]) + bta_ref[...]
    t = t_emb + t_attr

    WgT = WgT_ref[...]
    logits = (_mm(out, WgT[:D_OUT]) + _mm(t, WgT[D_OUT:2 * D_OUT])
              + _mm(t_attr, WgT[2 * D_OUT:]) + bg_ref[...])
    gate = 1.0 / (1.0 + jnp.exp(-logits))
    contrib = gate * t * mask_ref[...]

    o = out + contrib
    o_ref[...] = o

    @pl.when(pl.program_id(0) == 0)
    def _():
        sum_ref[...] = jnp.zeros_like(sum_ref)
        sq_ref[...] = jnp.zeros_like(sq_ref)

    sum_ref[...] += jnp.sum(o, axis=0, keepdims=True)
    sq_ref[...] += jnp.sum(o * o, axis=0, keepdims=True)


_row_spec = pl.BlockSpec((BM, D_IN), lambda i: (i, 0))
_half_spec = pl.BlockSpec((BM, D_HALF), lambda i: (i, 0))
_col_spec = pl.BlockSpec((BM, 1), lambda i: (i, 0))


def _w_spec(r, c):
    return pl.BlockSpec((r, c), lambda i: (0, 0))


_dense_a = pl.pallas_call(
    _dense_a_body,
    grid=(GRID,),
    in_specs=[
        _row_spec, _half_spec, _half_spec, _col_spec, _col_spec,
        pl.BlockSpec((BM, D_EDGE), lambda i: (i, 0)), _col_spec,
        _w_spec(D_IN, D_OUT), _w_spec(1, D_OUT), _w_spec(D_IN, D_OUT),
        _w_spec(D_EDGE, D_EDGE), _w_spec(1, D_EDGE),
        _w_spec(D_EDGE, D_OUT), _w_spec(1, D_OUT),
        _w_spec(D_EDGE, D_OUT), _w_spec(1, D_OUT),
        _w_spec(3 * D_OUT, D_OUT), _w_spec(1, D_OUT),
    ],
    out_specs=[
        pl.BlockSpec((BM, D_OUT), lambda i: (i, 0)),
        _w_spec(1, D_OUT), _w_spec(1, D_OUT),
    ],
    out_shape=[
        jax.ShapeDtypeStruct((N, D_OUT), jnp.float32),
        jax.ShapeDtypeStruct((1, D_OUT), jnp.float32),
        jax.ShapeDtypeStruct((1, D_OUT), jnp.float32),
    ],
)


def _dense_b_body(o_ref, sum_ref, sq_ref, gamma_ref, beta_ref, out_ref):
    mu = sum_ref[...] * (1.0 / N)
    var = sq_ref[...] * (1.0 / N) - mu * mu
    o = o_ref[...]
    o = (o - mu) / jnp.sqrt(var + 1e-5) * gamma_ref[...] + beta_ref[...]
    out_ref[...] = jnp.maximum(o + o, 0.0)


_dense_b = pl.pallas_call(
    _dense_b_body,
    grid=(GRID,),
    in_specs=[
        pl.BlockSpec((BM, D_OUT), lambda i: (i, 0)),
        _w_spec(1, D_OUT), _w_spec(1, D_OUT),
        _w_spec(1, D_OUT), _w_spec(1, D_OUT),
    ],
    out_specs=pl.BlockSpec((BM, D_OUT), lambda i: (i, 0)),
    out_shape=jax.ShapeDtypeStruct((N, D_OUT), jnp.float32),
)


def kernel(x, edge_index, edge_attr, W_l, b_l, W_r, W_emb, b_emb,
           W_te, b_te, W_ta, b_ta, W_gate, b_gate, gamma, beta):
    ei = edge_index.astype(jnp.int32).reshape(2, NS, NCHUNK, CHUNK)
    x2 = jnp.stack([x[:, :D_HALF], x[:, D_HALF:]])
    zrow = jnp.zeros((N_ACC, D_HALF), jnp.float32)
    zdeg = jnp.zeros((N_ACC,), jnp.float32)
    neg1 = jnp.full((N_PAD,), -1, jnp.int32)

    agg_p, deg_p, elast = _edge_pass(x2, ei, zrow, zdeg, neg1)
    mask, attr_sel = _select_pass(elast.reshape(NW, N_PAD), edge_attr)

    o, osum, osq = _dense_a(
        x, agg_p[0], agg_p[1],
        deg_p[0].reshape(N_ACC, 1), deg_p[1].reshape(N_ACC, 1),
        attr_sel, mask.reshape(N_PAD, 1),
        W_l.T, b_l.reshape(1, -1), W_r.T,
        W_emb.T, b_emb.reshape(1, -1), W_te.T, b_te.reshape(1, -1),
        W_ta.T, b_ta.reshape(1, -1),
        W_gate.T, b_gate.reshape(1, -1))
    return _dense_b(o, osum, osq, gamma.reshape(1, -1), beta.reshape(1, -1))
